# Initial kernel scaffold; baseline (speedup 1.0000x reference)
import jax
import jax.numpy as jnp
from jax.experimental import pallas as pl


def kernel(x, edge_index, edge_attr, batch, lin0_W, lin0_b, nn1_W, nn1_b, nn2_W, nn2_b, root_W, conv_b, gru_W_ih, gru_W_hh, gru_b_ih, gru_b_hh, lstm_W_ih, lstm_W_hh, lstm_b_ih, lstm_b_hh, lin1_W, lin1_b, lin2_W, lin2_b):
    raise NotImplementedError("write your pallas kernel here")


# speedup vs baseline: 1.4317x; 1.4317x over previous
import functools

import jax
import jax.numpy as jnp
from jax import lax
from jax.experimental import pallas as pl
from jax.experimental.pallas import tpu as pltpu
from jax.experimental.pallas import tpu_sc as plsc

N = 10000
E = 20000
FEAT = 16
D = 64
NG = 500

NP = 10240
EP = 20480
G = 512
NC = 2
NS = 16
NW = NC * NS
CH = 128
NCH = EP // (NW * CH)
RPT = NP // NS

EB = 512
NB = 512

def _sc_gather_body(nodes_hbm, src_hbm, xs_hbm, idx_v, row_v, sem):
    c = lax.axis_index("c")
    s = lax.axis_index("s")
    wid = c * NS + s
    pltpu.sync_copy(src_hbm.at[wid], idx_v)
    base = wid * NCH * CH
    for j in range(NCH):
        pltpu.async_copy(nodes_hbm.at[idx_v.at[j]], row_v, sem).wait()
        pltpu.sync_copy(row_v, xs_hbm.at[pl.ds(base + j * CH, CH)])


def _scatter_body(with_deg, msg_hbm, dst_hbm, z64_hbm, z16_hbm, ones_hbm,
                  agg_hbm, deg_hbm, idx_v, msg_v, ones_v, spA, spD):
    c = lax.axis_index("c")
    s = lax.axis_index("s")
    wid = c * NS + s
    rows = pl.ds(s * RPT, RPT)
    pltpu.sync_copy(z64_hbm.at[rows], spA.at[rows])
    if with_deg:
        pltpu.sync_copy(z16_hbm.at[rows], spD.at[rows])
        pltpu.sync_copy(ones_hbm, ones_v)
    plsc.subcore_barrier()
    pltpu.sync_copy(dst_hbm.at[wid], idx_v)
    base = wid * NCH * CH
    for j in range(NCH):
        pltpu.sync_copy(msg_hbm.at[pl.ds(base + j * CH, CH)], msg_v)
        pltpu.sync_copy(msg_v, spA.at[idx_v.at[j]], add=True)
        if with_deg:
            pltpu.sync_copy(ones_v, spD.at[idx_v.at[j]], add=True)
    plsc.subcore_barrier()
    pltpu.sync_copy(spA.at[rows], agg_hbm.at[c, rows])
    if with_deg:
        pltpu.sync_copy(spD.at[rows], deg_hbm.at[c, rows])


def _scatter_nodeg_body(msg_hbm, dst_hbm, z64_hbm, agg_hbm,
                        idx_v, msg_v, spA):
    _scatter_body(False, msg_hbm, dst_hbm, z64_hbm, None, None,
                  agg_hbm, None, idx_v, msg_v, None, spA, None)


@functools.cache
def _sc_kernels():
    mesh = plsc.VectorSubcoreMesh(
        core_axis_name="c", subcore_axis_name="s",
        num_cores=NC, num_subcores=NS)
    cp = pltpu.CompilerParams(use_tc_tiling_on_sc=False)
    gather = pl.kernel(
        _sc_gather_body,
        out_type=jax.ShapeDtypeStruct((EP, D), jnp.float32),
        mesh=mesh,
        scratch_types=[
            pltpu.VMEM((NCH, CH), jnp.int32),
            pltpu.VMEM((CH, D), jnp.float32),
            pltpu.SemaphoreType.DMA,
        ],
        compiler_params=cp,
    )
    scatter_deg = pl.kernel(
        functools.partial(_scatter_body, True),
        out_type=(jax.ShapeDtypeStruct((NC, NP, D), jnp.float32),
                  jax.ShapeDtypeStruct((NC, NP, 16), jnp.float32)),
        mesh=mesh,
        scratch_types=[
            pltpu.VMEM((NCH, CH), jnp.int32),
            pltpu.VMEM((CH, D), jnp.float32),
            pltpu.VMEM((CH, 16), jnp.float32),
            pltpu.VMEM_SHARED((NP, D), jnp.float32),
            pltpu.VMEM_SHARED((NP, 16), jnp.float32),
        ],
        compiler_params=cp,
    )
    scatter = pl.kernel(
        _scatter_nodeg_body,
        out_type=jax.ShapeDtypeStruct((NC, NP, D), jnp.float32),
        mesh=mesh,
        scratch_types=[
            pltpu.VMEM((NCH, CH), jnp.int32),
            pltpu.VMEM((CH, D), jnp.float32),
            pltpu.VMEM_SHARED((NP, D), jnp.float32),
        ],
        compiler_params=cp,
    )
    return gather, scatter_deg, scatter


def _lin0_body(x_ref, w_ref, b_ref, o_ref):
    o_ref[...] = jnp.maximum(x_ref[...] @ w_ref[...] + b_ref[...], 0.0)


_lin0 = pl.pallas_call(
    _lin0_body,
    grid=(NP // NB,),
    in_specs=[
        pl.BlockSpec((NB, FEAT), lambda i: (i, 0)),
        pl.BlockSpec((FEAT, D), lambda i: (0, 0)),
        pl.BlockSpec((1, D), lambda i: (0, 0)),
    ],
    out_specs=pl.BlockSpec((NB, D), lambda i: (i, 0)),
    out_shape=jax.ShapeDtypeStruct((NP, D), jnp.float32),
)


def _msg_body(ea_ref, xs_ref, w1_ref, b1_ref, w2_ref, b2_ref, o_ref,
              h2_ref, ew_ref):
    h2_ref[...] = jnp.maximum(ea_ref[...] @ w1_ref[...] + b1_ref[...], 0.0)
    acc = jnp.zeros((EB, D), jnp.float32)
    for cch in range(8):
        ew_ref[...] = (h2_ref[...] @ w2_ref[:, cch * 512:(cch + 1) * 512]
                       + b2_ref[:, cch * 512:(cch + 1) * 512])
        for i in range(8):
            ig = cch * 8 + i
            acc = acc + ew_ref[:, i * D:(i + 1) * D] * xs_ref[:, ig:ig + 1]
    o_ref[...] = acc


_msg = pl.pallas_call(
    _msg_body,
    grid=(EP // EB,),
    in_specs=[
        pl.BlockSpec((EB, 8), lambda i: (i, 0)),
        pl.BlockSpec((EB, D), lambda i: (i, 0)),
        pl.BlockSpec((8, 128), lambda i: (0, 0)),
        pl.BlockSpec((1, 128), lambda i: (0, 0)),
        pl.BlockSpec((128, 4096), lambda i: (0, 0)),
        pl.BlockSpec((1, 4096), lambda i: (0, 0)),
    ],
    out_specs=pl.BlockSpec((EB, D), lambda i: (i, 0)),
    out_shape=jax.ShapeDtypeStruct((EP, D), jnp.float32),
    scratch_shapes=[
        pltpu.VMEM((EB, 128), jnp.float32),
        pltpu.VMEM((EB, 512), jnp.float32),
    ],
)


def _sigmoid(x):
    return 1.0 / (1.0 + jnp.exp(-x))


def _gru_body(h_ref, a0_ref, a1_ref, d0_ref, d1_ref, rw_ref, cb_ref,
              wih_ref, whh_ref, bih_ref, bhh_ref, o_ref):
    h = h_ref[...]
    agg = a0_ref[...] + a1_ref[...]
    deg = jnp.maximum(d0_ref[:, 0:1] + d1_ref[:, 0:1], 1.0)
    m = jnp.maximum(agg / deg + h @ rw_ref[...] + cb_ref[...], 0.0)
    gi = m @ wih_ref[...] + bih_ref[...]
    gh = h @ whh_ref[...] + bhh_ref[...]
    r = _sigmoid(gi[:, 0:D] + gh[:, 0:D])
    z = _sigmoid(gi[:, D:2 * D] + gh[:, D:2 * D])
    cand = jnp.tanh(gi[:, 2 * D:3 * D] + r * gh[:, 2 * D:3 * D])
    o_ref[...] = (1.0 - z) * cand + z * h


_gru = pl.pallas_call(
    _gru_body,
    grid=(NP // NB,),
    in_specs=[
        pl.BlockSpec((NB, D), lambda i: (i, 0)),
        pl.BlockSpec((NB, D), lambda i: (i, 0)),
        pl.BlockSpec((NB, D), lambda i: (i, 0)),
        pl.BlockSpec((NB, 16), lambda i: (i, 0)),
        pl.BlockSpec((NB, 16), lambda i: (i, 0)),
        pl.BlockSpec((D, D), lambda i: (0, 0)),
        pl.BlockSpec((1, D), lambda i: (0, 0)),
        pl.BlockSpec((D, 3 * D), lambda i: (0, 0)),
        pl.BlockSpec((D, 3 * D), lambda i: (0, 0)),
        pl.BlockSpec((1, 3 * D), lambda i: (0, 0)),
        pl.BlockSpec((1, 3 * D), lambda i: (0, 0)),
    ],
    out_specs=pl.BlockSpec((NB, D), lambda i: (i, 0)),
    out_shape=jax.ShapeDtypeStruct((NP, D), jnp.float32),
)


def _s2s_b_body(out_ref, bcol_ref, q_ref, e_ref, emax_ref):
    k = pl.program_id(0)
    S = (bcol_ref[...] == lax.broadcasted_iota(
        jnp.int32, (NB, G), 1)).astype(jnp.float32)
    qe = S @ q_ref[...]
    e = jnp.sum(out_ref[...] * qe, axis=1, keepdims=True)
    e_ref[...] = e
    colvals = jnp.where(S > 0.5, e, -jnp.inf)
    pmax = jnp.max(colvals, axis=0, keepdims=True)

    @pl.when(k == 0)
    def _():
        emax_ref[...] = jnp.full((1, G), -jnp.inf, jnp.float32)

    emax_ref[...] = jnp.maximum(emax_ref[...], pmax)


_s2s_b = pl.pallas_call(
    _s2s_b_body,
    grid=(NP // NB,),
    in_specs=[
        pl.BlockSpec((NB, D), lambda i: (i, 0)),
        pl.BlockSpec((NB, 1), lambda i: (i, 0)),
        pl.BlockSpec((G, D), lambda i: (0, 0)),
    ],
    out_specs=[
        pl.BlockSpec((NB, 1), lambda i: (i, 0)),
        pl.BlockSpec((1, G), lambda i: (0, 0)),
    ],
    out_shape=[
        jax.ShapeDtypeStruct((NP, 1), jnp.float32),
        jax.ShapeDtypeStruct((1, G), jnp.float32),
    ],
)


def _s2s_c_body(out_ref, bcol_ref, brow_ref, e_ref, emax_ref,
                den_ref, rv_ref):
    k = pl.program_id(0)
    em = emax_ref[...]
    em = jnp.where(jnp.isfinite(em), em, 0.0)
    S = (bcol_ref[...] == lax.broadcasted_iota(
        jnp.int32, (NB, G), 1)).astype(jnp.float32)
    eexp = jnp.sum(S * em, axis=1, keepdims=True)
    ee = jnp.exp(e_ref[...] - eexp)
    ST = (brow_ref[...] == lax.broadcasted_iota(
        jnp.int32, (G, NB), 0)).astype(jnp.float32)

    @pl.when(k == 0)
    def _():
        den_ref[...] = jnp.zeros((G, 1), jnp.float32)
        rv_ref[...] = jnp.zeros((G, D), jnp.float32)

    den_ref[...] += ST @ ee
    rv_ref[...] += ST @ (ee * out_ref[...])


_s2s_c = pl.pallas_call(
    _s2s_c_body,
    grid=(NP // NB,),
    in_specs=[
        pl.BlockSpec((NB, D), lambda i: (i, 0)),
        pl.BlockSpec((NB, 1), lambda i: (i, 0)),
        pl.BlockSpec((1, NB), lambda i: (0, i)),
        pl.BlockSpec((NB, 1), lambda i: (i, 0)),
        pl.BlockSpec((1, G), lambda i: (0, 0)),
    ],
    out_specs=[
        pl.BlockSpec((G, 1), lambda i: (0, 0)),
        pl.BlockSpec((G, D), lambda i: (0, 0)),
    ],
    out_shape=[
        jax.ShapeDtypeStruct((G, 1), jnp.float32),
        jax.ShapeDtypeStruct((G, D), jnp.float32),
    ],
)


def _lstm_body(qp_ref, rvn_ref, den_ref, hs_ref, cs_ref, wq_ref, wr_ref,
               whh_ref, bih_ref, bhh_ref, hsn_ref, csn_ref):
    rvec = rvn_ref[...] / (den_ref[...] + 1e-16)
    gates = (qp_ref[...] @ wq_ref[...] + rvec @ wr_ref[...]
             + hs_ref[...] @ whh_ref[...] + bih_ref[...] + bhh_ref[...])
    ig = _sigmoid(gates[:, 0:D])
    fg = _sigmoid(gates[:, D:2 * D])
    gg = jnp.tanh(gates[:, 2 * D:3 * D])
    og = _sigmoid(gates[:, 3 * D:4 * D])
    cs = fg * cs_ref[...] + ig * gg
    hs = og * jnp.tanh(cs)
    hsn_ref[...] = hs
    csn_ref[...] = cs


_lstm = pl.pallas_call(
    _lstm_body,
    grid=(1,),
    in_specs=[
        pl.BlockSpec((G, D), lambda i: (0, 0)),
        pl.BlockSpec((G, D), lambda i: (0, 0)),
        pl.BlockSpec((G, 1), lambda i: (0, 0)),
        pl.BlockSpec((G, D), lambda i: (0, 0)),
        pl.BlockSpec((G, D), lambda i: (0, 0)),
        pl.BlockSpec((D, 4 * D), lambda i: (0, 0)),
        pl.BlockSpec((D, 4 * D), lambda i: (0, 0)),
        pl.BlockSpec((D, 4 * D), lambda i: (0, 0)),
        pl.BlockSpec((1, 4 * D), lambda i: (0, 0)),
        pl.BlockSpec((1, 4 * D), lambda i: (0, 0)),
    ],
    out_specs=[
        pl.BlockSpec((G, D), lambda i: (0, 0)),
        pl.BlockSpec((G, D), lambda i: (0, 0)),
    ],
    out_shape=[
        jax.ShapeDtypeStruct((G, D), jnp.float32),
        jax.ShapeDtypeStruct((G, D), jnp.float32),
    ],
)


def _head_body(q_ref, rvn_ref, den_ref, w1a_ref, w1b_ref, b1_ref,
               w2_ref, b2_ref, o_ref):
    rvec = rvn_ref[...] / (den_ref[...] + 1e-16)
    t = jnp.maximum(q_ref[...] @ w1a_ref[...] + rvec @ w1b_ref[...]
                    + b1_ref[...], 0.0)
    o_ref[...] = t @ w2_ref[...] + b2_ref[...]


_head = pl.pallas_call(
    _head_body,
    grid=(1,),
    in_specs=[
        pl.BlockSpec((G, D), lambda i: (0, 0)),
        pl.BlockSpec((G, D), lambda i: (0, 0)),
        pl.BlockSpec((G, 1), lambda i: (0, 0)),
        pl.BlockSpec((D, D), lambda i: (0, 0)),
        pl.BlockSpec((D, D), lambda i: (0, 0)),
        pl.BlockSpec((1, D), lambda i: (0, 0)),
        pl.BlockSpec((D, 1), lambda i: (0, 0)),
        pl.BlockSpec((1, 1), lambda i: (0, 0)),
    ],
    out_specs=pl.BlockSpec((G, 1), lambda i: (0, 0)),
    out_shape=jax.ShapeDtypeStruct((G, 1), jnp.float32),
)


def kernel(x, edge_index, edge_attr, batch, lin0_W, lin0_b, nn1_W, nn1_b,
           nn2_W, nn2_b, root_W, conv_b, gru_W_ih, gru_W_hh, gru_b_ih,
           gru_b_hh, lstm_W_ih, lstm_W_hh, lstm_b_ih, lstm_b_hh, lin1_W,
           lin1_b, lin2_W, lin2_b):
    f32 = jnp.float32
    xp = jnp.zeros((NP, FEAT), f32).at[:N].set(x)
    src = jnp.zeros((EP,), jnp.int32).at[:E].set(edge_index[0])
    dst = jnp.full((EP,), NP - 1, jnp.int32).at[:E].set(edge_index[1])
    src3 = src.reshape(NW, NCH, CH)
    dst3 = dst.reshape(NW, NCH, CH)
    ea8 = jnp.zeros((EP, 8), f32).at[:E, :5].set(edge_attr)
    batchp = jnp.full((NP,), G - 1, jnp.int32).at[:N].set(batch)
    bcol = batchp.reshape(NP, 1)
    brow = batchp.reshape(1, NP)
    w1p = jnp.zeros((8, 128), f32).at[:5].set(nn1_W)
    z64 = jnp.zeros((NP, D), f32)
    z16 = jnp.zeros((NP, 16), f32)
    ones_ch = jnp.ones((CH, 16), f32)

    sc_gather, sc_scatter_deg, sc_scatter = _sc_kernels()
    h = _lin0(xp, lin0_W, lin0_b.reshape(1, D))
    degp = None
    for it in range(3):
        xs = sc_gather(h, src3)
        msg = _msg(ea8, xs, w1p, nn1_b.reshape(1, 128), nn2_W,
                   nn2_b.reshape(1, 4096))
        if it == 0:
            aggp, degp = sc_scatter_deg(msg, dst3, z64, z16, ones_ch)
        else:
            aggp = sc_scatter(msg, dst3, z64)
        h = _gru(h, aggp[0], aggp[1], degp[0], degp[1], root_W,
                 conv_b.reshape(1, D), gru_W_ih, gru_W_hh,
                 gru_b_ih.reshape(1, 3 * D), gru_b_hh.reshape(1, 3 * D))

    out = h
    wq = lstm_W_ih[:D]
    wr = lstm_W_ih[D:]
    qprev = jnp.zeros((G, D), f32)
    rvn = jnp.zeros((G, D), f32)
    den = jnp.ones((G, 1), f32)
    hs = jnp.zeros((G, D), f32)
    cs = jnp.zeros((G, D), f32)
    for it in range(3):
        hs, cs = _lstm(qprev, rvn, den, hs, cs, wq, wr, lstm_W_hh,
                       lstm_b_ih.reshape(1, 4 * D),
                       lstm_b_hh.reshape(1, 4 * D))
        e, emax = _s2s_b(out, bcol, hs)
        den, rvn = _s2s_c(out, bcol, brow, e, emax)
        qprev = hs
    o = _head(qprev, rvn, den, lin1_W[:D], lin1_W[D:],
              lin1_b.reshape(1, D), lin2_W, lin2_b.reshape(1, 1))
    return o[:NG, 0]


# speedup vs baseline: 2.1258x; 1.4848x over previous
import functools

import jax
import jax.numpy as jnp
from jax import lax
from jax.experimental import pallas as pl
from jax.experimental.pallas import tpu as pltpu
from jax.experimental.pallas import tpu_sc as plsc

N = 10000
E = 20000
FEAT = 16
D = 64
NG = 500

NP = 10240
EP = 20480
G = 512
NC = 2
NS = 16
NW = NC * NS
CH = 128
NCH = EP // (NW * CH)
RPT = NP // NS

EB = 512
NB = 512

def _sc_gather_body(nodes_hbm, src_hbm, xs_hbm, idx_v, row_v, sem):
    c = lax.axis_index("c")
    s = lax.axis_index("s")
    wid = c * NS + s
    pltpu.sync_copy(src_hbm.at[wid], idx_v)
    base = wid * NCH * CH
    for j in range(NCH):
        pltpu.async_copy(nodes_hbm.at[idx_v.at[j]], row_v, sem).wait()
        pltpu.sync_copy(row_v, xs_hbm.at[pl.ds(base + j * CH, CH)])


def _scatter_body(with_deg, msg_hbm, dst_hbm, z64_hbm, z16_hbm, ones_hbm,
                  agg_hbm, deg_hbm, idx_v, msg_v, ones_v, spA, spD):
    c = lax.axis_index("c")
    s = lax.axis_index("s")
    wid = c * NS + s
    rows = pl.ds(s * RPT, RPT)
    pltpu.sync_copy(z64_hbm.at[rows], spA.at[rows])
    if with_deg:
        pltpu.sync_copy(z16_hbm.at[rows], spD.at[rows])
        pltpu.sync_copy(ones_hbm, ones_v)
    plsc.subcore_barrier()
    pltpu.sync_copy(dst_hbm.at[wid], idx_v)
    base = wid * NCH * CH
    for j in range(NCH):
        pltpu.sync_copy(msg_hbm.at[pl.ds(base + j * CH, CH)], msg_v)
        pltpu.sync_copy(msg_v, spA.at[idx_v.at[j]], add=True)
        if with_deg:
            pltpu.sync_copy(ones_v, spD.at[idx_v.at[j]], add=True)
    plsc.subcore_barrier()
    pltpu.sync_copy(spA.at[rows], agg_hbm.at[c, rows])
    if with_deg:
        pltpu.sync_copy(spD.at[rows], deg_hbm.at[c, rows])


def _scatter_nodeg_body(msg_hbm, dst_hbm, z64_hbm, agg_hbm,
                        idx_v, msg_v, spA):
    _scatter_body(False, msg_hbm, dst_hbm, z64_hbm, None, None,
                  agg_hbm, None, idx_v, msg_v, None, spA, None)


@functools.cache
def _sc_kernels():
    mesh = plsc.VectorSubcoreMesh(
        core_axis_name="c", subcore_axis_name="s",
        num_cores=NC, num_subcores=NS)
    cp = pltpu.CompilerParams(use_tc_tiling_on_sc=False)
    gather = pl.kernel(
        _sc_gather_body,
        out_type=jax.ShapeDtypeStruct((EP, D), jnp.float32),
        mesh=mesh,
        scratch_types=[
            pltpu.VMEM((NCH, CH), jnp.int32),
            pltpu.VMEM((CH, D), jnp.float32),
            pltpu.SemaphoreType.DMA,
        ],
        compiler_params=cp,
    )
    scatter_deg = pl.kernel(
        functools.partial(_scatter_body, True),
        out_type=(jax.ShapeDtypeStruct((NC, NP, D), jnp.float32),
                  jax.ShapeDtypeStruct((NC, NP, 16), jnp.float32)),
        mesh=mesh,
        scratch_types=[
            pltpu.VMEM((NCH, CH), jnp.int32),
            pltpu.VMEM((CH, D), jnp.float32),
            pltpu.VMEM((CH, 16), jnp.float32),
            pltpu.VMEM_SHARED((NP, D), jnp.float32),
            pltpu.VMEM_SHARED((NP, 16), jnp.float32),
        ],
        compiler_params=cp,
    )
    scatter = pl.kernel(
        _scatter_nodeg_body,
        out_type=jax.ShapeDtypeStruct((NC, NP, D), jnp.float32),
        mesh=mesh,
        scratch_types=[
            pltpu.VMEM((NCH, CH), jnp.int32),
            pltpu.VMEM((CH, D), jnp.float32),
            pltpu.VMEM_SHARED((NP, D), jnp.float32),
        ],
        compiler_params=cp,
    )
    return gather, scatter_deg, scatter


def _lin0_body(x_ref, w_ref, b_ref, o_ref):
    o_ref[...] = jnp.maximum(x_ref[...] @ w_ref[...] + b_ref[...], 0.0)


_lin0 = pl.pallas_call(
    _lin0_body,
    grid=(NP // NB,),
    in_specs=[
        pl.BlockSpec((NB, FEAT), lambda i: (i, 0)),
        pl.BlockSpec((FEAT, D), lambda i: (0, 0)),
        pl.BlockSpec((1, D), lambda i: (0, 0)),
    ],
    out_specs=pl.BlockSpec((NB, D), lambda i: (i, 0)),
    out_shape=jax.ShapeDtypeStruct((NP, D), jnp.float32),
)


def _msg_body(ea_ref, xs_ref, w1_ref, b1_ref, w2_ref, b2_ref, r_ref, m_ref,
              o_ref, h2_ref, ew_ref):
    h2_ref[...] = jnp.maximum(ea_ref[...] @ w1_ref[...] + b1_ref[...], 0.0)
    acc = jnp.zeros((EB, D), jnp.float32)
    for cch in range(8):
        ew_ref[...] = (h2_ref[...] @ w2_ref[:, cch * 512:(cch + 1) * 512]
                       + b2_ref[:, cch * 512:(cch + 1) * 512])
        xb = xs_ref[...] @ r_ref[:, cch * 512:(cch + 1) * 512]
        acc = acc + (ew_ref[...] * xb) @ m_ref[...]
    o_ref[...] = acc


_msg = pl.pallas_call(
    _msg_body,
    grid=(EP // EB,),
    in_specs=[
        pl.BlockSpec((EB, 8), lambda i: (i, 0)),
        pl.BlockSpec((EB, D), lambda i: (i, 0)),
        pl.BlockSpec((8, 128), lambda i: (0, 0)),
        pl.BlockSpec((1, 128), lambda i: (0, 0)),
        pl.BlockSpec((128, 4096), lambda i: (0, 0)),
        pl.BlockSpec((1, 4096), lambda i: (0, 0)),
        pl.BlockSpec((D, 4096), lambda i: (0, 0)),
        pl.BlockSpec((512, D), lambda i: (0, 0)),
    ],
    out_specs=pl.BlockSpec((EB, D), lambda i: (i, 0)),
    out_shape=jax.ShapeDtypeStruct((EP, D), jnp.float32),
    scratch_shapes=[
        pltpu.VMEM((EB, 128), jnp.float32),
        pltpu.VMEM((EB, 512), jnp.float32),
    ],
)


def _sigmoid(x):
    return 1.0 / (1.0 + jnp.exp(-x))


def _gru_body(h_ref, a0_ref, a1_ref, d0_ref, d1_ref, rw_ref, cb_ref,
              wih_ref, whh_ref, bih_ref, bhh_ref, o_ref):
    h = h_ref[...]
    agg = a0_ref[...] + a1_ref[...]
    deg = jnp.maximum(d0_ref[:, 0:1] + d1_ref[:, 0:1], 1.0)
    m = jnp.maximum(agg / deg + h @ rw_ref[...] + cb_ref[...], 0.0)
    gi = m @ wih_ref[...] + bih_ref[...]
    gh = h @ whh_ref[...] + bhh_ref[...]
    r = _sigmoid(gi[:, 0:D] + gh[:, 0:D])
    z = _sigmoid(gi[:, D:2 * D] + gh[:, D:2 * D])
    cand = jnp.tanh(gi[:, 2 * D:3 * D] + r * gh[:, 2 * D:3 * D])
    o_ref[...] = (1.0 - z) * cand + z * h


_gru = pl.pallas_call(
    _gru_body,
    grid=(NP // NB,),
    in_specs=[
        pl.BlockSpec((NB, D), lambda i: (i, 0)),
        pl.BlockSpec((NB, D), lambda i: (i, 0)),
        pl.BlockSpec((NB, D), lambda i: (i, 0)),
        pl.BlockSpec((NB, 16), lambda i: (i, 0)),
        pl.BlockSpec((NB, 16), lambda i: (i, 0)),
        pl.BlockSpec((D, D), lambda i: (0, 0)),
        pl.BlockSpec((1, D), lambda i: (0, 0)),
        pl.BlockSpec((D, 3 * D), lambda i: (0, 0)),
        pl.BlockSpec((D, 3 * D), lambda i: (0, 0)),
        pl.BlockSpec((1, 3 * D), lambda i: (0, 0)),
        pl.BlockSpec((1, 3 * D), lambda i: (0, 0)),
    ],
    out_specs=pl.BlockSpec((NB, D), lambda i: (i, 0)),
    out_shape=jax.ShapeDtypeStruct((NP, D), jnp.float32),
)


def _s2s_b_body(out_ref, bcol_ref, q_ref, e_ref, emax_ref):
    k = pl.program_id(0)
    S = (bcol_ref[...] == lax.broadcasted_iota(
        jnp.int32, (NB, G), 1)).astype(jnp.float32)
    qe = S @ q_ref[...]
    e = jnp.sum(out_ref[...] * qe, axis=1, keepdims=True)
    e_ref[...] = e
    colvals = jnp.where(S > 0.5, e, -jnp.inf)
    pmax = jnp.max(colvals, axis=0, keepdims=True)

    @pl.when(k == 0)
    def _():
        emax_ref[...] = jnp.full((1, G), -jnp.inf, jnp.float32)

    emax_ref[...] = jnp.maximum(emax_ref[...], pmax)


_s2s_b = pl.pallas_call(
    _s2s_b_body,
    grid=(NP // NB,),
    in_specs=[
        pl.BlockSpec((NB, D), lambda i: (i, 0)),
        pl.BlockSpec((NB, 1), lambda i: (i, 0)),
        pl.BlockSpec((G, D), lambda i: (0, 0)),
    ],
    out_specs=[
        pl.BlockSpec((NB, 1), lambda i: (i, 0)),
        pl.BlockSpec((1, G), lambda i: (0, 0)),
    ],
    out_shape=[
        jax.ShapeDtypeStruct((NP, 1), jnp.float32),
        jax.ShapeDtypeStruct((1, G), jnp.float32),
    ],
)


def _s2s_c_body(out_ref, bcol_ref, brow_ref, e_ref, emax_ref,
                den_ref, rv_ref):
    k = pl.program_id(0)
    em = emax_ref[...]
    em = jnp.where(jnp.isfinite(em), em, 0.0)
    S = (bcol_ref[...] == lax.broadcasted_iota(
        jnp.int32, (NB, G), 1)).astype(jnp.float32)
    eexp = jnp.sum(S * em, axis=1, keepdims=True)
    ee = jnp.exp(e_ref[...] - eexp)
    ST = (brow_ref[...] == lax.broadcasted_iota(
        jnp.int32, (G, NB), 0)).astype(jnp.float32)

    @pl.when(k == 0)
    def _():
        den_ref[...] = jnp.zeros((G, 1), jnp.float32)
        rv_ref[...] = jnp.zeros((G, D), jnp.float32)

    den_ref[...] += ST @ ee
    rv_ref[...] += ST @ (ee * out_ref[...])


_s2s_c = pl.pallas_call(
    _s2s_c_body,
    grid=(NP // NB,),
    in_specs=[
        pl.BlockSpec((NB, D), lambda i: (i, 0)),
        pl.BlockSpec((NB, 1), lambda i: (i, 0)),
        pl.BlockSpec((1, NB), lambda i: (0, i)),
        pl.BlockSpec((NB, 1), lambda i: (i, 0)),
        pl.BlockSpec((1, G), lambda i: (0, 0)),
    ],
    out_specs=[
        pl.BlockSpec((G, 1), lambda i: (0, 0)),
        pl.BlockSpec((G, D), lambda i: (0, 0)),
    ],
    out_shape=[
        jax.ShapeDtypeStruct((G, 1), jnp.float32),
        jax.ShapeDtypeStruct((G, D), jnp.float32),
    ],
)


def _lstm_body(qp_ref, rvn_ref, den_ref, hs_ref, cs_ref, wq_ref, wr_ref,
               whh_ref, bih_ref, bhh_ref, hsn_ref, csn_ref):
    rvec = rvn_ref[...] / (den_ref[...] + 1e-16)
    gates = (qp_ref[...] @ wq_ref[...] + rvec @ wr_ref[...]
             + hs_ref[...] @ whh_ref[...] + bih_ref[...] + bhh_ref[...])
    ig = _sigmoid(gates[:, 0:D])
    fg = _sigmoid(gates[:, D:2 * D])
    gg = jnp.tanh(gates[:, 2 * D:3 * D])
    og = _sigmoid(gates[:, 3 * D:4 * D])
    cs = fg * cs_ref[...] + ig * gg
    hs = og * jnp.tanh(cs)
    hsn_ref[...] = hs
    csn_ref[...] = cs


_lstm = pl.pallas_call(
    _lstm_body,
    grid=(1,),
    in_specs=[
        pl.BlockSpec((G, D), lambda i: (0, 0)),
        pl.BlockSpec((G, D), lambda i: (0, 0)),
        pl.BlockSpec((G, 1), lambda i: (0, 0)),
        pl.BlockSpec((G, D), lambda i: (0, 0)),
        pl.BlockSpec((G, D), lambda i: (0, 0)),
        pl.BlockSpec((D, 4 * D), lambda i: (0, 0)),
        pl.BlockSpec((D, 4 * D), lambda i: (0, 0)),
        pl.BlockSpec((D, 4 * D), lambda i: (0, 0)),
        pl.BlockSpec((1, 4 * D), lambda i: (0, 0)),
        pl.BlockSpec((1, 4 * D), lambda i: (0, 0)),
    ],
    out_specs=[
        pl.BlockSpec((G, D), lambda i: (0, 0)),
        pl.BlockSpec((G, D), lambda i: (0, 0)),
    ],
    out_shape=[
        jax.ShapeDtypeStruct((G, D), jnp.float32),
        jax.ShapeDtypeStruct((G, D), jnp.float32),
    ],
)


def _head_body(q_ref, rvn_ref, den_ref, w1a_ref, w1b_ref, b1_ref,
               w2_ref, b2_ref, o_ref):
    rvec = rvn_ref[...] / (den_ref[...] + 1e-16)
    t = jnp.maximum(q_ref[...] @ w1a_ref[...] + rvec @ w1b_ref[...]
                    + b1_ref[...], 0.0)
    o_ref[...] = t @ w2_ref[...] + b2_ref[...]


_head = pl.pallas_call(
    _head_body,
    grid=(1,),
    in_specs=[
        pl.BlockSpec((G, D), lambda i: (0, 0)),
        pl.BlockSpec((G, D), lambda i: (0, 0)),
        pl.BlockSpec((G, 1), lambda i: (0, 0)),
        pl.BlockSpec((D, D), lambda i: (0, 0)),
        pl.BlockSpec((D, D), lambda i: (0, 0)),
        pl.BlockSpec((1, D), lambda i: (0, 0)),
        pl.BlockSpec((D, 1), lambda i: (0, 0)),
        pl.BlockSpec((1, 1), lambda i: (0, 0)),
    ],
    out_specs=pl.BlockSpec((G, 1), lambda i: (0, 0)),
    out_shape=jax.ShapeDtypeStruct((G, 1), jnp.float32),
)


def kernel(x, edge_index, edge_attr, batch, lin0_W, lin0_b, nn1_W, nn1_b,
           nn2_W, nn2_b, root_W, conv_b, gru_W_ih, gru_W_hh, gru_b_ih,
           gru_b_hh, lstm_W_ih, lstm_W_hh, lstm_b_ih, lstm_b_hh, lin1_W,
           lin1_b, lin2_W, lin2_b):
    f32 = jnp.float32
    xp = jnp.zeros((NP, FEAT), f32).at[:N].set(x)
    src = jnp.zeros((EP,), jnp.int32).at[:E].set(edge_index[0])
    dst = jnp.full((EP,), NP - 1, jnp.int32).at[:E].set(edge_index[1])
    src3 = src.reshape(NW, NCH, CH)
    dst3 = dst.reshape(NW, NCH, CH)
    ea8 = jnp.zeros((EP, 8), f32).at[:E, :5].set(edge_attr)
    batchp = jnp.full((NP,), G - 1, jnp.int32).at[:N].set(batch)
    bcol = batchp.reshape(NP, 1)
    brow = batchp.reshape(1, NP)
    w1p = jnp.zeros((8, 128), f32).at[:5].set(nn1_W)
    z64 = jnp.zeros((NP, D), f32)
    z16 = jnp.zeros((NP, 16), f32)
    ones_ch = jnp.ones((CH, 16), f32)
    lane = jnp.arange(4096, dtype=jnp.int32)
    rmat = (lane[None, :] // D == jnp.arange(D, dtype=jnp.int32)[:, None]
            ).astype(f32)
    mmat = (lane[:512, None] % D == jnp.arange(D, dtype=jnp.int32)[None, :]
            ).astype(f32)

    sc_gather, sc_scatter_deg, sc_scatter = _sc_kernels()
    h = _lin0(xp, lin0_W, lin0_b.reshape(1, D))
    degp = None
    for it in range(3):
        xs = sc_gather(h, src3)
        msg = _msg(ea8, xs, w1p, nn1_b.reshape(1, 128), nn2_W,
                   nn2_b.reshape(1, 4096), rmat, mmat)
        if it == 0:
            aggp, degp = sc_scatter_deg(msg, dst3, z64, z16, ones_ch)
        else:
            aggp = sc_scatter(msg, dst3, z64)
        h = _gru(h, aggp[0], aggp[1], degp[0], degp[1], root_W,
                 conv_b.reshape(1, D), gru_W_ih, gru_W_hh,
                 gru_b_ih.reshape(1, 3 * D), gru_b_hh.reshape(1, 3 * D))

    out = h
    wq = lstm_W_ih[:D]
    wr = lstm_W_ih[D:]
    qprev = jnp.zeros((G, D), f32)
    rvn = jnp.zeros((G, D), f32)
    den = jnp.ones((G, 1), f32)
    hs = jnp.zeros((G, D), f32)
    cs = jnp.zeros((G, D), f32)
    for it in range(3):
        hs, cs = _lstm(qprev, rvn, den, hs, cs, wq, wr, lstm_W_hh,
                       lstm_b_ih.reshape(1, 4 * D),
                       lstm_b_hh.reshape(1, 4 * D))
        e, emax = _s2s_b(out, bcol, hs)
        den, rvn = _s2s_c(out, bcol, brow, e, emax)
        qprev = hs
    o = _head(qprev, rvn, den, lin1_W[:D], lin1_W[D:],
              lin1_b.reshape(1, D), lin2_W, lin2_b.reshape(1, 1))
    return o[:NG, 0]


# speedup vs baseline: 2.4134x; 1.1353x over previous
import functools

import jax
import jax.numpy as jnp
from jax import lax
from jax.experimental import pallas as pl
from jax.experimental.pallas import tpu as pltpu
from jax.experimental.pallas import tpu_sc as plsc

N = 10000
E = 20000
FEAT = 16
D = 64
NG = 500

NP = 10240
EP = 20480
G = 512
NC = 2
NS = 16
NW = NC * NS
CH = 128
NCH = EP // (NW * CH)
RPT = NP // NS

EB = 512
NB = 512

def _sc_gather_body(nodes_hbm, src_hbm, xs_hbm, idx_v, row_v, sem):
    c = lax.axis_index("c")
    s = lax.axis_index("s")
    wid = c * NS + s
    pltpu.sync_copy(src_hbm.at[wid], idx_v)
    base = wid * NCH * CH
    for j in range(NCH):
        pltpu.async_copy(nodes_hbm.at[idx_v.at[j]], row_v, sem).wait()
        pltpu.sync_copy(row_v, xs_hbm.at[pl.ds(base + j * CH, CH)])


def _scatter_body(with_deg, msg_hbm, dst_hbm, z64_hbm, z16_hbm, ones_hbm,
                  agg_hbm, deg_hbm, idx_v, msg_v, ones_v, spA, spD):
    c = lax.axis_index("c")
    s = lax.axis_index("s")
    wid = c * NS + s
    rows = pl.ds(s * RPT, RPT)
    pltpu.sync_copy(z64_hbm.at[rows], spA.at[rows])
    if with_deg:
        pltpu.sync_copy(z16_hbm.at[rows], spD.at[rows])
        pltpu.sync_copy(ones_hbm, ones_v)
    plsc.subcore_barrier()
    pltpu.sync_copy(dst_hbm.at[wid], idx_v)
    base = wid * NCH * CH
    for j in range(NCH):
        pltpu.sync_copy(msg_hbm.at[pl.ds(base + j * CH, CH)], msg_v)
        pltpu.sync_copy(msg_v, spA.at[idx_v.at[j]], add=True)
        if with_deg:
            pltpu.sync_copy(ones_v, spD.at[idx_v.at[j]], add=True)
    plsc.subcore_barrier()
    pltpu.sync_copy(spA.at[rows], agg_hbm.at[c, rows])
    if with_deg:
        pltpu.sync_copy(spD.at[rows], deg_hbm.at[c, rows])


def _scatter_nodeg_body(msg_hbm, dst_hbm, z64_hbm, agg_hbm,
                        idx_v, msg_v, spA):
    _scatter_body(False, msg_hbm, dst_hbm, z64_hbm, None, None,
                  agg_hbm, None, idx_v, msg_v, None, spA, None)


@functools.cache
def _sc_kernels():
    mesh = plsc.VectorSubcoreMesh(
        core_axis_name="c", subcore_axis_name="s",
        num_cores=NC, num_subcores=NS)
    cp = pltpu.CompilerParams(use_tc_tiling_on_sc=False)
    gather = pl.kernel(
        _sc_gather_body,
        out_type=jax.ShapeDtypeStruct((EP, D), jnp.float32),
        mesh=mesh,
        scratch_types=[
            pltpu.VMEM((NCH, CH), jnp.int32),
            pltpu.VMEM((CH, D), jnp.float32),
            pltpu.SemaphoreType.DMA,
        ],
        compiler_params=cp,
    )
    scatter_deg = pl.kernel(
        functools.partial(_scatter_body, True),
        out_type=(jax.ShapeDtypeStruct((NC, NP, D), jnp.float32),
                  jax.ShapeDtypeStruct((NC, NP, 16), jnp.float32)),
        mesh=mesh,
        scratch_types=[
            pltpu.VMEM((NCH, CH), jnp.int32),
            pltpu.VMEM((CH, D), jnp.float32),
            pltpu.VMEM((CH, 16), jnp.float32),
            pltpu.VMEM_SHARED((NP, D), jnp.float32),
            pltpu.VMEM_SHARED((NP, 16), jnp.float32),
        ],
        compiler_params=cp,
    )
    scatter = pl.kernel(
        _scatter_nodeg_body,
        out_type=jax.ShapeDtypeStruct((NC, NP, D), jnp.float32),
        mesh=mesh,
        scratch_types=[
            pltpu.VMEM((NCH, CH), jnp.int32),
            pltpu.VMEM((CH, D), jnp.float32),
            pltpu.VMEM_SHARED((NP, D), jnp.float32),
        ],
        compiler_params=cp,
    )
    return gather, scatter_deg, scatter


def _lin0_body(x_ref, w_ref, b_ref, o_ref):
    o_ref[...] = jnp.maximum(x_ref[...] @ w_ref[...] + b_ref[...], 0.0)


_lin0 = pl.pallas_call(
    _lin0_body,
    grid=(NP // NB,),
    in_specs=[
        pl.BlockSpec((NB, FEAT), lambda i: (i, 0)),
        pl.BlockSpec((FEAT, D), lambda i: (0, 0)),
        pl.BlockSpec((1, D), lambda i: (0, 0)),
    ],
    out_specs=pl.BlockSpec((NB, D), lambda i: (i, 0)),
    out_shape=jax.ShapeDtypeStruct((NP, D), jnp.float32),
)


def _msg_body(ea_ref, xs_ref, w1_ref, b1_ref, w2_ref, b2_ref, r_ref, m_ref,
              o_ref, h2_ref, ew_ref):
    h2_ref[...] = jnp.maximum(ea_ref[...] @ w1_ref[...] + b1_ref[...], 0.0)
    for cch in range(8):
        ew = (h2_ref[...] @ w2_ref[:, cch * 512:(cch + 1) * 512]
              + b2_ref[:, cch * 512:(cch + 1) * 512])
        xb = xs_ref[...] @ r_ref[:, cch * 512:(cch + 1) * 512]
        if cch == 0:
            ew_ref[...] = ew * xb
        else:
            ew_ref[...] += ew * xb
    o_ref[...] = ew_ref[...] @ m_ref[...]


_msg = pl.pallas_call(
    _msg_body,
    grid=(EP // EB,),
    in_specs=[
        pl.BlockSpec((EB, 8), lambda i: (i, 0)),
        pl.BlockSpec((EB, D), lambda i: (i, 0)),
        pl.BlockSpec((8, 128), lambda i: (0, 0)),
        pl.BlockSpec((1, 128), lambda i: (0, 0)),
        pl.BlockSpec((128, 4096), lambda i: (0, 0)),
        pl.BlockSpec((1, 4096), lambda i: (0, 0)),
        pl.BlockSpec((D, 4096), lambda i: (0, 0)),
        pl.BlockSpec((512, D), lambda i: (0, 0)),
    ],
    out_specs=pl.BlockSpec((EB, D), lambda i: (i, 0)),
    out_shape=jax.ShapeDtypeStruct((EP, D), jnp.float32),
    scratch_shapes=[
        pltpu.VMEM((EB, 128), jnp.float32),
        pltpu.VMEM((EB, 512), jnp.float32),
    ],
)


def _sigmoid(x):
    return 1.0 / (1.0 + jnp.exp(-x))


def _gru_body(h_ref, a0_ref, a1_ref, d0_ref, d1_ref, rw_ref, cb_ref,
              wih_ref, whh_ref, bih_ref, bhh_ref, o_ref):
    h = h_ref[...]
    agg = a0_ref[...] + a1_ref[...]
    deg = jnp.maximum(d0_ref[:, 0:1] + d1_ref[:, 0:1], 1.0)
    m = jnp.maximum(agg / deg + h @ rw_ref[...] + cb_ref[...], 0.0)
    gi = m @ wih_ref[...] + bih_ref[...]
    gh = h @ whh_ref[...] + bhh_ref[...]
    r = _sigmoid(gi[:, 0:D] + gh[:, 0:D])
    z = _sigmoid(gi[:, D:2 * D] + gh[:, D:2 * D])
    cand = jnp.tanh(gi[:, 2 * D:3 * D] + r * gh[:, 2 * D:3 * D])
    o_ref[...] = (1.0 - z) * cand + z * h


_gru = pl.pallas_call(
    _gru_body,
    grid=(NP // NB,),
    in_specs=[
        pl.BlockSpec((NB, D), lambda i: (i, 0)),
        pl.BlockSpec((NB, D), lambda i: (i, 0)),
        pl.BlockSpec((NB, D), lambda i: (i, 0)),
        pl.BlockSpec((NB, 16), lambda i: (i, 0)),
        pl.BlockSpec((NB, 16), lambda i: (i, 0)),
        pl.BlockSpec((D, D), lambda i: (0, 0)),
        pl.BlockSpec((1, D), lambda i: (0, 0)),
        pl.BlockSpec((D, 3 * D), lambda i: (0, 0)),
        pl.BlockSpec((D, 3 * D), lambda i: (0, 0)),
        pl.BlockSpec((1, 3 * D), lambda i: (0, 0)),
        pl.BlockSpec((1, 3 * D), lambda i: (0, 0)),
    ],
    out_specs=pl.BlockSpec((NB, D), lambda i: (i, 0)),
    out_shape=jax.ShapeDtypeStruct((NP, D), jnp.float32),
)


def _s2s_b_body(out_ref, bcol_ref, q_ref, e_ref, emax_ref):
    k = pl.program_id(0)
    S = (bcol_ref[...] == lax.broadcasted_iota(
        jnp.int32, (NB, G), 1)).astype(jnp.float32)
    qe = S @ q_ref[...]
    e = jnp.sum(out_ref[...] * qe, axis=1, keepdims=True)
    e_ref[...] = e
    colvals = jnp.where(S > 0.5, e, -jnp.inf)
    pmax = jnp.max(colvals, axis=0, keepdims=True)

    @pl.when(k == 0)
    def _():
        emax_ref[...] = jnp.full((1, G), -jnp.inf, jnp.float32)

    emax_ref[...] = jnp.maximum(emax_ref[...], pmax)


_s2s_b = pl.pallas_call(
    _s2s_b_body,
    grid=(NP // NB,),
    in_specs=[
        pl.BlockSpec((NB, D), lambda i: (i, 0)),
        pl.BlockSpec((NB, 1), lambda i: (i, 0)),
        pl.BlockSpec((G, D), lambda i: (0, 0)),
    ],
    out_specs=[
        pl.BlockSpec((NB, 1), lambda i: (i, 0)),
        pl.BlockSpec((1, G), lambda i: (0, 0)),
    ],
    out_shape=[
        jax.ShapeDtypeStruct((NP, 1), jnp.float32),
        jax.ShapeDtypeStruct((1, G), jnp.float32),
    ],
)


def _s2s_c_body(out_ref, bcol_ref, brow_ref, e_ref, emax_ref,
                den_ref, rv_ref):
    k = pl.program_id(0)
    em = emax_ref[...]
    em = jnp.where(jnp.isfinite(em), em, 0.0)
    S = (bcol_ref[...] == lax.broadcasted_iota(
        jnp.int32, (NB, G), 1)).astype(jnp.float32)
    eexp = jnp.sum(S * em, axis=1, keepdims=True)
    ee = jnp.exp(e_ref[...] - eexp)
    ST = (brow_ref[...] == lax.broadcasted_iota(
        jnp.int32, (G, NB), 0)).astype(jnp.float32)

    @pl.when(k == 0)
    def _():
        den_ref[...] = jnp.zeros((G, 1), jnp.float32)
        rv_ref[...] = jnp.zeros((G, D), jnp.float32)

    den_ref[...] += ST @ ee
    rv_ref[...] += ST @ (ee * out_ref[...])


_s2s_c = pl.pallas_call(
    _s2s_c_body,
    grid=(NP // NB,),
    in_specs=[
        pl.BlockSpec((NB, D), lambda i: (i, 0)),
        pl.BlockSpec((NB, 1), lambda i: (i, 0)),
        pl.BlockSpec((1, NB), lambda i: (0, i)),
        pl.BlockSpec((NB, 1), lambda i: (i, 0)),
        pl.BlockSpec((1, G), lambda i: (0, 0)),
    ],
    out_specs=[
        pl.BlockSpec((G, 1), lambda i: (0, 0)),
        pl.BlockSpec((G, D), lambda i: (0, 0)),
    ],
    out_shape=[
        jax.ShapeDtypeStruct((G, 1), jnp.float32),
        jax.ShapeDtypeStruct((G, D), jnp.float32),
    ],
)


def _lstm_body(qp_ref, rvn_ref, den_ref, hs_ref, cs_ref, wq_ref, wr_ref,
               whh_ref, bih_ref, bhh_ref, hsn_ref, csn_ref):
    rvec = rvn_ref[...] / (den_ref[...] + 1e-16)
    gates = (qp_ref[...] @ wq_ref[...] + rvec @ wr_ref[...]
             + hs_ref[...] @ whh_ref[...] + bih_ref[...] + bhh_ref[...])
    ig = _sigmoid(gates[:, 0:D])
    fg = _sigmoid(gates[:, D:2 * D])
    gg = jnp.tanh(gates[:, 2 * D:3 * D])
    og = _sigmoid(gates[:, 3 * D:4 * D])
    cs = fg * cs_ref[...] + ig * gg
    hs = og * jnp.tanh(cs)
    hsn_ref[...] = hs
    csn_ref[...] = cs


_lstm = pl.pallas_call(
    _lstm_body,
    grid=(1,),
    in_specs=[
        pl.BlockSpec((G, D), lambda i: (0, 0)),
        pl.BlockSpec((G, D), lambda i: (0, 0)),
        pl.BlockSpec((G, 1), lambda i: (0, 0)),
        pl.BlockSpec((G, D), lambda i: (0, 0)),
        pl.BlockSpec((G, D), lambda i: (0, 0)),
        pl.BlockSpec((D, 4 * D), lambda i: (0, 0)),
        pl.BlockSpec((D, 4 * D), lambda i: (0, 0)),
        pl.BlockSpec((D, 4 * D), lambda i: (0, 0)),
        pl.BlockSpec((1, 4 * D), lambda i: (0, 0)),
        pl.BlockSpec((1, 4 * D), lambda i: (0, 0)),
    ],
    out_specs=[
        pl.BlockSpec((G, D), lambda i: (0, 0)),
        pl.BlockSpec((G, D), lambda i: (0, 0)),
    ],
    out_shape=[
        jax.ShapeDtypeStruct((G, D), jnp.float32),
        jax.ShapeDtypeStruct((G, D), jnp.float32),
    ],
)


def _head_body(q_ref, rvn_ref, den_ref, w1a_ref, w1b_ref, b1_ref,
               w2_ref, b2_ref, o_ref):
    rvec = rvn_ref[...] / (den_ref[...] + 1e-16)
    t = jnp.maximum(q_ref[...] @ w1a_ref[...] + rvec @ w1b_ref[...]
                    + b1_ref[...], 0.0)
    o_ref[...] = t @ w2_ref[...] + b2_ref[...]


_head = pl.pallas_call(
    _head_body,
    grid=(1,),
    in_specs=[
        pl.BlockSpec((G, D), lambda i: (0, 0)),
        pl.BlockSpec((G, D), lambda i: (0, 0)),
        pl.BlockSpec((G, 1), lambda i: (0, 0)),
        pl.BlockSpec((D, D), lambda i: (0, 0)),
        pl.BlockSpec((D, D), lambda i: (0, 0)),
        pl.BlockSpec((1, D), lambda i: (0, 0)),
        pl.BlockSpec((D, 1), lambda i: (0, 0)),
        pl.BlockSpec((1, 1), lambda i: (0, 0)),
    ],
    out_specs=pl.BlockSpec((G, 1), lambda i: (0, 0)),
    out_shape=jax.ShapeDtypeStruct((G, 1), jnp.float32),
)


def kernel(x, edge_index, edge_attr, batch, lin0_W, lin0_b, nn1_W, nn1_b,
           nn2_W, nn2_b, root_W, conv_b, gru_W_ih, gru_W_hh, gru_b_ih,
           gru_b_hh, lstm_W_ih, lstm_W_hh, lstm_b_ih, lstm_b_hh, lin1_W,
           lin1_b, lin2_W, lin2_b):
    f32 = jnp.float32
    xp = jnp.zeros((NP, FEAT), f32).at[:N].set(x)
    src = jnp.zeros((EP,), jnp.int32).at[:E].set(edge_index[0])
    dst = jnp.full((EP,), NP - 1, jnp.int32).at[:E].set(edge_index[1])
    src3 = src.reshape(NW, NCH, CH)
    dst3 = dst.reshape(NW, NCH, CH)
    ea8 = jnp.zeros((EP, 8), f32).at[:E, :5].set(edge_attr)
    batchp = jnp.full((NP,), G - 1, jnp.int32).at[:N].set(batch)
    bcol = batchp.reshape(NP, 1)
    brow = batchp.reshape(1, NP)
    w1p = jnp.zeros((8, 128), f32).at[:5].set(nn1_W)
    z64 = jnp.zeros((NP, D), f32)
    z16 = jnp.zeros((NP, 16), f32)
    ones_ch = jnp.ones((CH, 16), f32)
    lane = jnp.arange(4096, dtype=jnp.int32)
    rmat = (lane[None, :] // D == jnp.arange(D, dtype=jnp.int32)[:, None]
            ).astype(f32)
    mmat = (lane[:512, None] % D == jnp.arange(D, dtype=jnp.int32)[None, :]
            ).astype(f32)

    sc_gather, sc_scatter_deg, sc_scatter = _sc_kernels()
    h = _lin0(xp, lin0_W, lin0_b.reshape(1, D))
    degp = None
    for it in range(3):
        xs = sc_gather(h, src3)
        msg = _msg(ea8, xs, w1p, nn1_b.reshape(1, 128), nn2_W,
                   nn2_b.reshape(1, 4096), rmat, mmat)
        if it == 0:
            aggp, degp = sc_scatter_deg(msg, dst3, z64, z16, ones_ch)
        else:
            aggp = sc_scatter(msg, dst3, z64)
        h = _gru(h, aggp[0], aggp[1], degp[0], degp[1], root_W,
                 conv_b.reshape(1, D), gru_W_ih, gru_W_hh,
                 gru_b_ih.reshape(1, 3 * D), gru_b_hh.reshape(1, 3 * D))

    out = h
    wq = lstm_W_ih[:D]
    wr = lstm_W_ih[D:]
    qprev = jnp.zeros((G, D), f32)
    rvn = jnp.zeros((G, D), f32)
    den = jnp.ones((G, 1), f32)
    hs = jnp.zeros((G, D), f32)
    cs = jnp.zeros((G, D), f32)
    for it in range(3):
        hs, cs = _lstm(qprev, rvn, den, hs, cs, wq, wr, lstm_W_hh,
                       lstm_b_ih.reshape(1, 4 * D),
                       lstm_b_hh.reshape(1, 4 * D))
        e, emax = _s2s_b(out, bcol, hs)
        den, rvn = _s2s_c(out, bcol, brow, e, emax)
        qprev = hs
    o = _head(qprev, rvn, den, lin1_W[:D], lin1_W[D:],
              lin1_b.reshape(1, D), lin2_W, lin2_b.reshape(1, 1))
    return o[:NG, 0]


# speedup vs baseline: 2.5099x; 1.0400x over previous
import functools

import jax
import jax.numpy as jnp
from jax import lax
from jax.experimental import pallas as pl
from jax.experimental.pallas import tpu as pltpu
from jax.experimental.pallas import tpu_sc as plsc

N = 10000
E = 20000
FEAT = 16
D = 64
NG = 500

NP = 10240
EP = 20480
G = 512
NC = 2
NS = 16
NW = NC * NS
CH = 128
NCH = EP // (NW * CH)
RPT = NP // NS

EB = 512
NB = 512

def _sc_gather_body(nodes_hbm, src_hbm, xs_hbm, idx_v, row_v, gsem, wsem):
    c = lax.axis_index("c")
    s = lax.axis_index("s")
    wid = c * NS + s
    pltpu.sync_copy(src_hbm.at[wid], idx_v)
    base = wid * NCH * CH
    gd = [pltpu.async_copy(nodes_hbm.at[idx_v.at[j]], row_v.at[j], gsem)
          for j in range(NCH)]
    for d in gd:
        d.wait()
    wd = [pltpu.async_copy(row_v.at[j],
                           xs_hbm.at[pl.ds(base + j * CH, CH)], wsem)
          for j in range(NCH)]
    for d in wd:
        d.wait()


def _scatter_body(with_deg, msg_hbm, dst_hbm, z64_hbm, z16_hbm, ones_hbm,
                  agg_hbm, deg_hbm, idx_v, msg_v, ones_v, spA, spD,
                  lsem, ssem):
    c = lax.axis_index("c")
    s = lax.axis_index("s")
    wid = c * NS + s
    rows = pl.ds(s * RPT, RPT)
    base = wid * NCH * CH
    pltpu.sync_copy(dst_hbm.at[wid], idx_v)
    ld = [pltpu.async_copy(msg_hbm.at[pl.ds(base + j * CH, CH)],
                           msg_v.at[j], lsem)
          for j in range(NCH)]
    pltpu.sync_copy(z64_hbm.at[rows], spA.at[rows])
    if with_deg:
        pltpu.sync_copy(z16_hbm.at[rows], spD.at[rows])
        pltpu.sync_copy(ones_hbm, ones_v)
    for d in ld:
        d.wait()
    plsc.subcore_barrier()
    sd = [pltpu.async_copy(msg_v.at[j], spA.at[idx_v.at[j]], ssem, add=True)
          for j in range(NCH)]
    if with_deg:
        sd += [pltpu.async_copy(ones_v, spD.at[idx_v.at[j]], ssem, add=True)
               for j in range(NCH)]
    for d in sd:
        d.wait()
    plsc.subcore_barrier()
    pltpu.sync_copy(spA.at[rows], agg_hbm.at[c, rows])
    if with_deg:
        pltpu.sync_copy(spD.at[rows], deg_hbm.at[c, rows])


def _scatter_nodeg_body(msg_hbm, dst_hbm, z64_hbm, agg_hbm,
                        idx_v, msg_v, spA, lsem, ssem):
    _scatter_body(False, msg_hbm, dst_hbm, z64_hbm, None, None,
                  agg_hbm, None, idx_v, msg_v, None, spA, None, lsem, ssem)


@functools.cache
def _sc_kernels():
    mesh = plsc.VectorSubcoreMesh(
        core_axis_name="c", subcore_axis_name="s",
        num_cores=NC, num_subcores=NS)
    cp = pltpu.CompilerParams(use_tc_tiling_on_sc=False)
    gather = pl.kernel(
        _sc_gather_body,
        out_type=jax.ShapeDtypeStruct((EP, D), jnp.float32),
        mesh=mesh,
        scratch_types=[
            pltpu.VMEM((NCH, CH), jnp.int32),
            pltpu.VMEM((NCH, CH, D), jnp.float32),
            pltpu.SemaphoreType.DMA,
            pltpu.SemaphoreType.DMA,
        ],
        compiler_params=cp,
    )
    scatter_deg = pl.kernel(
        functools.partial(_scatter_body, True),
        out_type=(jax.ShapeDtypeStruct((NC, NP, D), jnp.float32),
                  jax.ShapeDtypeStruct((NC, NP, 16), jnp.float32)),
        mesh=mesh,
        scratch_types=[
            pltpu.VMEM((NCH, CH), jnp.int32),
            pltpu.VMEM((NCH, CH, D), jnp.float32),
            pltpu.VMEM((CH, 16), jnp.float32),
            pltpu.VMEM_SHARED((NP, D), jnp.float32),
            pltpu.VMEM_SHARED((NP, 16), jnp.float32),
            pltpu.SemaphoreType.DMA,
            pltpu.SemaphoreType.DMA,
        ],
        compiler_params=cp,
    )
    scatter = pl.kernel(
        _scatter_nodeg_body,
        out_type=jax.ShapeDtypeStruct((NC, NP, D), jnp.float32),
        mesh=mesh,
        scratch_types=[
            pltpu.VMEM((NCH, CH), jnp.int32),
            pltpu.VMEM((NCH, CH, D), jnp.float32),
            pltpu.VMEM_SHARED((NP, D), jnp.float32),
            pltpu.SemaphoreType.DMA,
            pltpu.SemaphoreType.DMA,
        ],
        compiler_params=cp,
    )
    return gather, scatter_deg, scatter


def _lin0_body(x_ref, w_ref, b_ref, o_ref):
    o_ref[...] = jnp.maximum(x_ref[...] @ w_ref[...] + b_ref[...], 0.0)


_lin0 = pl.pallas_call(
    _lin0_body,
    grid=(NP // NB,),
    in_specs=[
        pl.BlockSpec((NB, FEAT), lambda i: (i, 0)),
        pl.BlockSpec((FEAT, D), lambda i: (0, 0)),
        pl.BlockSpec((1, D), lambda i: (0, 0)),
    ],
    out_specs=pl.BlockSpec((NB, D), lambda i: (i, 0)),
    out_shape=jax.ShapeDtypeStruct((NP, D), jnp.float32),
)


def _msg_body(ea_ref, xs_ref, w1_ref, b1_ref, w2_ref, b2_ref, r_ref, m_ref,
              o_ref, h2_ref, ew_ref):
    h2_ref[...] = jnp.maximum(ea_ref[...] @ w1_ref[...] + b1_ref[...], 0.0)
    for cch in range(8):
        ew = (h2_ref[...] @ w2_ref[:, cch * 512:(cch + 1) * 512]
              + b2_ref[:, cch * 512:(cch + 1) * 512])
        xb = xs_ref[...] @ r_ref[:, cch * 512:(cch + 1) * 512]
        if cch == 0:
            ew_ref[...] = ew * xb
        else:
            ew_ref[...] += ew * xb
    o_ref[...] = ew_ref[...] @ m_ref[...]


_msg = pl.pallas_call(
    _msg_body,
    grid=(EP // EB,),
    in_specs=[
        pl.BlockSpec((EB, 8), lambda i: (i, 0)),
        pl.BlockSpec((EB, D), lambda i: (i, 0)),
        pl.BlockSpec((8, 128), lambda i: (0, 0)),
        pl.BlockSpec((1, 128), lambda i: (0, 0)),
        pl.BlockSpec((128, 4096), lambda i: (0, 0)),
        pl.BlockSpec((1, 4096), lambda i: (0, 0)),
        pl.BlockSpec((D, 4096), lambda i: (0, 0)),
        pl.BlockSpec((512, D), lambda i: (0, 0)),
    ],
    out_specs=pl.BlockSpec((EB, D), lambda i: (i, 0)),
    out_shape=jax.ShapeDtypeStruct((EP, D), jnp.float32),
    scratch_shapes=[
        pltpu.VMEM((EB, 128), jnp.float32),
        pltpu.VMEM((EB, 512), jnp.float32),
    ],
)


def _sigmoid(x):
    return 1.0 / (1.0 + jnp.exp(-x))


def _gru_body(h_ref, a0_ref, a1_ref, d0_ref, d1_ref, rw_ref, cb_ref,
              wih_ref, whh_ref, bih_ref, bhh_ref, o_ref):
    h = h_ref[...]
    agg = a0_ref[...] + a1_ref[...]
    deg = jnp.maximum(d0_ref[:, 0:1] + d1_ref[:, 0:1], 1.0)
    m = jnp.maximum(agg / deg + h @ rw_ref[...] + cb_ref[...], 0.0)
    gi = m @ wih_ref[...] + bih_ref[...]
    gh = h @ whh_ref[...] + bhh_ref[...]
    r = _sigmoid(gi[:, 0:D] + gh[:, 0:D])
    z = _sigmoid(gi[:, D:2 * D] + gh[:, D:2 * D])
    cand = jnp.tanh(gi[:, 2 * D:3 * D] + r * gh[:, 2 * D:3 * D])
    o_ref[...] = (1.0 - z) * cand + z * h


_gru = pl.pallas_call(
    _gru_body,
    grid=(NP // NB,),
    in_specs=[
        pl.BlockSpec((NB, D), lambda i: (i, 0)),
        pl.BlockSpec((NB, D), lambda i: (i, 0)),
        pl.BlockSpec((NB, D), lambda i: (i, 0)),
        pl.BlockSpec((NB, 16), lambda i: (i, 0)),
        pl.BlockSpec((NB, 16), lambda i: (i, 0)),
        pl.BlockSpec((D, D), lambda i: (0, 0)),
        pl.BlockSpec((1, D), lambda i: (0, 0)),
        pl.BlockSpec((D, 3 * D), lambda i: (0, 0)),
        pl.BlockSpec((D, 3 * D), lambda i: (0, 0)),
        pl.BlockSpec((1, 3 * D), lambda i: (0, 0)),
        pl.BlockSpec((1, 3 * D), lambda i: (0, 0)),
    ],
    out_specs=pl.BlockSpec((NB, D), lambda i: (i, 0)),
    out_shape=jax.ShapeDtypeStruct((NP, D), jnp.float32),
)


def _s2s_body(out_ref, bcol_ref, brow_ref, qp_ref, rvn_ref, den_ref,
              hs_ref, cs_ref, wq_ref, wr_ref, whh_ref, bih_ref, bhh_ref,
              hsn_ref, csn_ref, dout_ref, rvout_ref, q_scr, e_scr, emax_scr):
    p = pl.program_id(0)
    k = pl.program_id(1)

    @pl.when((p == 0) & (k == 0))
    def _():
        rvec = rvn_ref[...] / (den_ref[...] + 1e-16)
        gates = (qp_ref[...] @ wq_ref[...] + rvec @ wr_ref[...]
                 + hs_ref[...] @ whh_ref[...] + bih_ref[...] + bhh_ref[...])
        ig = _sigmoid(gates[:, 0:D])
        fg = _sigmoid(gates[:, D:2 * D])
        gg = jnp.tanh(gates[:, 2 * D:3 * D])
        og = _sigmoid(gates[:, 3 * D:4 * D])
        cs = fg * cs_ref[...] + ig * gg
        hs = og * jnp.tanh(cs)
        hsn_ref[...] = hs
        csn_ref[...] = cs
        q_scr[...] = hs
        emax_scr[...] = jnp.full((1, G), -jnp.inf, jnp.float32)

    S = (bcol_ref[...] == lax.broadcasted_iota(
        jnp.int32, (NB, G), 1)).astype(jnp.float32)

    @pl.when(p == 0)
    def _():
        qe = S @ q_scr[...]
        e = jnp.sum(out_ref[...] * qe, axis=1, keepdims=True)
        e_scr[pl.ds(k * NB, NB), :] = e
        colvals = jnp.where(S > 0.5, e, -jnp.inf)
        pmax = jnp.max(colvals, axis=0, keepdims=True)
        emax_scr[...] = jnp.maximum(emax_scr[...], pmax)

    @pl.when(p == 1)
    def _():
        em = emax_scr[...]
        em = jnp.where(jnp.isfinite(em), em, 0.0)
        eexp = jnp.sum(S * em, axis=1, keepdims=True)
        ee = jnp.exp(e_scr[pl.ds(k * NB, NB), :] - eexp)
        ST = (brow_ref[...] == lax.broadcasted_iota(
            jnp.int32, (G, NB), 0)).astype(jnp.float32)

        @pl.when(k == 0)
        def _():
            dout_ref[...] = jnp.zeros((G, 1), jnp.float32)
            rvout_ref[...] = jnp.zeros((G, D), jnp.float32)

        dout_ref[...] += ST @ ee
        rvout_ref[...] += ST @ (ee * out_ref[...])


_s2s = pl.pallas_call(
    _s2s_body,
    grid=(2, NP // NB),
    in_specs=[
        pl.BlockSpec((NB, D), lambda p, k: (k, 0)),
        pl.BlockSpec((NB, 1), lambda p, k: (k, 0)),
        pl.BlockSpec((1, NB), lambda p, k: (0, k)),
        pl.BlockSpec((G, D), lambda p, k: (0, 0)),
        pl.BlockSpec((G, D), lambda p, k: (0, 0)),
        pl.BlockSpec((G, 1), lambda p, k: (0, 0)),
        pl.BlockSpec((G, D), lambda p, k: (0, 0)),
        pl.BlockSpec((G, D), lambda p, k: (0, 0)),
        pl.BlockSpec((D, 4 * D), lambda p, k: (0, 0)),
        pl.BlockSpec((D, 4 * D), lambda p, k: (0, 0)),
        pl.BlockSpec((D, 4 * D), lambda p, k: (0, 0)),
        pl.BlockSpec((1, 4 * D), lambda p, k: (0, 0)),
        pl.BlockSpec((1, 4 * D), lambda p, k: (0, 0)),
    ],
    out_specs=[
        pl.BlockSpec((G, D), lambda p, k: (0, 0)),
        pl.BlockSpec((G, D), lambda p, k: (0, 0)),
        pl.BlockSpec((G, 1), lambda p, k: (0, 0)),
        pl.BlockSpec((G, D), lambda p, k: (0, 0)),
    ],
    out_shape=[
        jax.ShapeDtypeStruct((G, D), jnp.float32),
        jax.ShapeDtypeStruct((G, D), jnp.float32),
        jax.ShapeDtypeStruct((G, 1), jnp.float32),
        jax.ShapeDtypeStruct((G, D), jnp.float32),
    ],
    scratch_shapes=[
        pltpu.VMEM((G, D), jnp.float32),
        pltpu.VMEM((NP, 1), jnp.float32),
        pltpu.VMEM((1, G), jnp.float32),
    ],
)


def _head_body(q_ref, rvn_ref, den_ref, w1a_ref, w1b_ref, b1_ref,
               w2_ref, b2_ref, o_ref):
    rvec = rvn_ref[...] / (den_ref[...] + 1e-16)
    t = jnp.maximum(q_ref[...] @ w1a_ref[...] + rvec @ w1b_ref[...]
                    + b1_ref[...], 0.0)
    o_ref[...] = t @ w2_ref[...] + b2_ref[...]


_head = pl.pallas_call(
    _head_body,
    grid=(1,),
    in_specs=[
        pl.BlockSpec((G, D), lambda i: (0, 0)),
        pl.BlockSpec((G, D), lambda i: (0, 0)),
        pl.BlockSpec((G, 1), lambda i: (0, 0)),
        pl.BlockSpec((D, D), lambda i: (0, 0)),
        pl.BlockSpec((D, D), lambda i: (0, 0)),
        pl.BlockSpec((1, D), lambda i: (0, 0)),
        pl.BlockSpec((D, 1), lambda i: (0, 0)),
        pl.BlockSpec((1, 1), lambda i: (0, 0)),
    ],
    out_specs=pl.BlockSpec((G, 1), lambda i: (0, 0)),
    out_shape=jax.ShapeDtypeStruct((G, 1), jnp.float32),
)


def kernel(x, edge_index, edge_attr, batch, lin0_W, lin0_b, nn1_W, nn1_b,
           nn2_W, nn2_b, root_W, conv_b, gru_W_ih, gru_W_hh, gru_b_ih,
           gru_b_hh, lstm_W_ih, lstm_W_hh, lstm_b_ih, lstm_b_hh, lin1_W,
           lin1_b, lin2_W, lin2_b):
    f32 = jnp.float32
    xp = jnp.zeros((NP, FEAT), f32).at[:N].set(x)
    src = jnp.zeros((EP,), jnp.int32).at[:E].set(edge_index[0])
    dst = jnp.full((EP,), NP - 1, jnp.int32).at[:E].set(edge_index[1])
    src3 = src.reshape(NW, NCH, CH)
    dst3 = dst.reshape(NW, NCH, CH)
    ea8 = jnp.zeros((EP, 8), f32).at[:E, :5].set(edge_attr)
    batchp = jnp.full((NP,), G - 1, jnp.int32).at[:N].set(batch)
    bcol = batchp.reshape(NP, 1)
    brow = batchp.reshape(1, NP)
    w1p = jnp.zeros((8, 128), f32).at[:5].set(nn1_W)
    z64 = jnp.zeros((NP, D), f32)
    z16 = jnp.zeros((NP, 16), f32)
    ones_ch = jnp.ones((CH, 16), f32)
    lane = jnp.arange(4096, dtype=jnp.int32)
    rmat = (lane[None, :] // D == jnp.arange(D, dtype=jnp.int32)[:, None]
            ).astype(f32)
    mmat = (lane[:512, None] % D == jnp.arange(D, dtype=jnp.int32)[None, :]
            ).astype(f32)

    sc_gather, sc_scatter_deg, sc_scatter = _sc_kernels()
    h = _lin0(xp, lin0_W, lin0_b.reshape(1, D))
    degp = None
    for it in range(3):
        xs = sc_gather(h, src3)
        msg = _msg(ea8, xs, w1p, nn1_b.reshape(1, 128), nn2_W,
                   nn2_b.reshape(1, 4096), rmat, mmat)
        if it == 0:
            aggp, degp = sc_scatter_deg(msg, dst3, z64, z16, ones_ch)
        else:
            aggp = sc_scatter(msg, dst3, z64)
        h = _gru(h, aggp[0], aggp[1], degp[0], degp[1], root_W,
                 conv_b.reshape(1, D), gru_W_ih, gru_W_hh,
                 gru_b_ih.reshape(1, 3 * D), gru_b_hh.reshape(1, 3 * D))

    out = h
    wq = lstm_W_ih[:D]
    wr = lstm_W_ih[D:]
    qprev = jnp.zeros((G, D), f32)
    rvn = jnp.zeros((G, D), f32)
    den = jnp.ones((G, 1), f32)
    hs = jnp.zeros((G, D), f32)
    cs = jnp.zeros((G, D), f32)
    for it in range(3):
        hs, cs, den, rvn = _s2s(out, bcol, brow, qprev, rvn, den, hs, cs,
                                wq, wr, lstm_W_hh,
                                lstm_b_ih.reshape(1, 4 * D),
                                lstm_b_hh.reshape(1, 4 * D))
        qprev = hs
    o = _head(qprev, rvn, den, lin1_W[:D], lin1_W[D:],
              lin1_b.reshape(1, D), lin2_W, lin2_b.reshape(1, 1))
    return o[:NG, 0]


# speedup vs baseline: 2.5306x; 1.0082x over previous
import functools

import jax
import jax.numpy as jnp
from jax import lax
from jax.experimental import pallas as pl
from jax.experimental.pallas import tpu as pltpu
from jax.experimental.pallas import tpu_sc as plsc

N = 10000
E = 20000
FEAT = 16
D = 64
NG = 500

NP = 10240
EP = 20480
G = 512
NC = 2
NS = 16
NW = NC * NS
CH = 128
NCH = EP // (NW * CH)
RPT = NP // NS

EB = 512
NB = 512

def _sc_gather_body(nodes_hbm, src_hbm, xs_hbm, idx_v, row_v, gsem, wsem):
    c = lax.axis_index("c")
    s = lax.axis_index("s")
    wid = c * NS + s
    pltpu.sync_copy(src_hbm.at[wid], idx_v)
    base = wid * NCH * CH
    gd = [pltpu.async_copy(nodes_hbm.at[idx_v.at[j]], row_v.at[j], gsem)
          for j in range(NCH)]
    for d in gd:
        d.wait()
    wd = [pltpu.async_copy(row_v.at[j],
                           xs_hbm.at[pl.ds(base + j * CH, CH)], wsem)
          for j in range(NCH)]
    for d in wd:
        d.wait()


def _scatter_body(with_deg, msg_hbm, dst_hbm, z64_hbm, z16_hbm, ones_hbm,
                  agg_hbm, deg_hbm, idx_v, msg_v, ones_v, spA, spD,
                  lsem, ssem):
    c = lax.axis_index("c")
    s = lax.axis_index("s")
    wid = c * NS + s
    rows = pl.ds(s * RPT, RPT)
    base = wid * NCH * CH
    pltpu.sync_copy(dst_hbm.at[wid], idx_v)
    ld = [pltpu.async_copy(msg_hbm.at[pl.ds(base + j * CH, CH)],
                           msg_v.at[j], lsem)
          for j in range(NCH)]
    pltpu.sync_copy(z64_hbm.at[rows], spA.at[rows])
    if with_deg:
        pltpu.sync_copy(z16_hbm.at[rows], spD.at[rows])
        pltpu.sync_copy(ones_hbm, ones_v)
    for d in ld:
        d.wait()
    plsc.subcore_barrier()
    sd = [pltpu.async_copy(msg_v.at[j], spA.at[idx_v.at[j]], ssem, add=True)
          for j in range(NCH)]
    if with_deg:
        sd += [pltpu.async_copy(ones_v, spD.at[idx_v.at[j]], ssem, add=True)
               for j in range(NCH)]
    for d in sd:
        d.wait()
    plsc.subcore_barrier()
    pltpu.sync_copy(spA.at[rows], agg_hbm.at[c, rows])
    if with_deg:
        pltpu.sync_copy(spD.at[rows], deg_hbm.at[c, rows])


def _scatter_nodeg_body(msg_hbm, dst_hbm, z64_hbm, agg_hbm,
                        idx_v, msg_v, spA, lsem, ssem):
    _scatter_body(False, msg_hbm, dst_hbm, z64_hbm, None, None,
                  agg_hbm, None, idx_v, msg_v, None, spA, None, lsem, ssem)


@functools.cache
def _sc_kernels():
    mesh = plsc.VectorSubcoreMesh(
        core_axis_name="c", subcore_axis_name="s",
        num_cores=NC, num_subcores=NS)
    cp = pltpu.CompilerParams(use_tc_tiling_on_sc=False)
    gather = pl.kernel(
        _sc_gather_body,
        out_type=jax.ShapeDtypeStruct((EP, D), jnp.float32),
        mesh=mesh,
        scratch_types=[
            pltpu.VMEM((NCH, CH), jnp.int32),
            pltpu.VMEM((NCH, CH, D), jnp.float32),
            pltpu.SemaphoreType.DMA,
            pltpu.SemaphoreType.DMA,
        ],
        compiler_params=cp,
    )
    scatter_deg = pl.kernel(
        functools.partial(_scatter_body, True),
        out_type=(jax.ShapeDtypeStruct((NC, NP, D), jnp.float32),
                  jax.ShapeDtypeStruct((NC, NP, 16), jnp.float32)),
        mesh=mesh,
        scratch_types=[
            pltpu.VMEM((NCH, CH), jnp.int32),
            pltpu.VMEM((NCH, CH, D), jnp.float32),
            pltpu.VMEM((CH, 16), jnp.float32),
            pltpu.VMEM_SHARED((NP, D), jnp.float32),
            pltpu.VMEM_SHARED((NP, 16), jnp.float32),
            pltpu.SemaphoreType.DMA,
            pltpu.SemaphoreType.DMA,
        ],
        compiler_params=cp,
    )
    scatter = pl.kernel(
        _scatter_nodeg_body,
        out_type=jax.ShapeDtypeStruct((NC, NP, D), jnp.float32),
        mesh=mesh,
        scratch_types=[
            pltpu.VMEM((NCH, CH), jnp.int32),
            pltpu.VMEM((NCH, CH, D), jnp.float32),
            pltpu.VMEM_SHARED((NP, D), jnp.float32),
            pltpu.SemaphoreType.DMA,
            pltpu.SemaphoreType.DMA,
        ],
        compiler_params=cp,
    )
    return gather, scatter_deg, scatter


def _lin0_body(x_ref, w_ref, b_ref, o_ref):
    o_ref[...] = jnp.maximum(x_ref[...] @ w_ref[...] + b_ref[...], 0.0)


_lin0 = pl.pallas_call(
    _lin0_body,
    grid=(NP // NB,),
    in_specs=[
        pl.BlockSpec((NB, FEAT), lambda i: (i, 0)),
        pl.BlockSpec((FEAT, D), lambda i: (0, 0)),
        pl.BlockSpec((1, D), lambda i: (0, 0)),
    ],
    out_specs=pl.BlockSpec((NB, D), lambda i: (i, 0)),
    out_shape=jax.ShapeDtypeStruct((NP, D), jnp.float32),
)


def _msg_body(ea_ref, xs_ref, w1_ref, b1_ref, w2_ref, b2_ref, r_ref, m_ref,
              o_ref, h2_ref, ew_ref):
    h2_ref[...] = jnp.maximum(ea_ref[...] @ w1_ref[...] + b1_ref[...], 0.0)
    for cch in range(8):
        ew = (h2_ref[...] @ w2_ref[:, cch * 512:(cch + 1) * 512]
              + b2_ref[:, cch * 512:(cch + 1) * 512])
        xb = xs_ref[...] @ r_ref[:, cch * 512:(cch + 1) * 512]
        if cch == 0:
            ew_ref[...] = ew * xb
        else:
            ew_ref[...] += ew * xb
    o_ref[...] = ew_ref[...] @ m_ref[...]


_msg = pl.pallas_call(
    _msg_body,
    grid=(EP // EB,),
    in_specs=[
        pl.BlockSpec((EB, 8), lambda i: (i, 0)),
        pl.BlockSpec((EB, D), lambda i: (i, 0)),
        pl.BlockSpec((8, 128), lambda i: (0, 0)),
        pl.BlockSpec((1, 128), lambda i: (0, 0)),
        pl.BlockSpec((128, 4096), lambda i: (0, 0)),
        pl.BlockSpec((1, 4096), lambda i: (0, 0)),
        pl.BlockSpec((D, 4096), lambda i: (0, 0)),
        pl.BlockSpec((512, D), lambda i: (0, 0)),
    ],
    out_specs=pl.BlockSpec((EB, D), lambda i: (i, 0)),
    out_shape=jax.ShapeDtypeStruct((EP, D), jnp.float32),
    scratch_shapes=[
        pltpu.VMEM((EB, 128), jnp.float32),
        pltpu.VMEM((EB, 512), jnp.float32),
    ],
)


def _sigmoid(x):
    return 1.0 / (1.0 + jnp.exp(-x))


def _gru_body(h_ref, a0_ref, a1_ref, d0_ref, d1_ref, rw_ref, cb_ref,
              wih_ref, whh_ref, bih_ref, bhh_ref, o_ref):
    h = h_ref[...]
    agg = a0_ref[...] + a1_ref[...]
    deg = jnp.maximum(d0_ref[:, 0:1] + d1_ref[:, 0:1], 1.0)
    m = jnp.maximum(agg / deg + h @ rw_ref[...] + cb_ref[...], 0.0)
    gi = m @ wih_ref[...] + bih_ref[...]
    gh = h @ whh_ref[...] + bhh_ref[...]
    r = _sigmoid(gi[:, 0:D] + gh[:, 0:D])
    z = _sigmoid(gi[:, D:2 * D] + gh[:, D:2 * D])
    cand = jnp.tanh(gi[:, 2 * D:3 * D] + r * gh[:, 2 * D:3 * D])
    o_ref[...] = (1.0 - z) * cand + z * h


_gru = pl.pallas_call(
    _gru_body,
    grid=(NP // NB,),
    in_specs=[
        pl.BlockSpec((NB, D), lambda i: (i, 0)),
        pl.BlockSpec((NB, D), lambda i: (i, 0)),
        pl.BlockSpec((NB, D), lambda i: (i, 0)),
        pl.BlockSpec((NB, 16), lambda i: (i, 0)),
        pl.BlockSpec((NB, 16), lambda i: (i, 0)),
        pl.BlockSpec((D, D), lambda i: (0, 0)),
        pl.BlockSpec((1, D), lambda i: (0, 0)),
        pl.BlockSpec((D, 3 * D), lambda i: (0, 0)),
        pl.BlockSpec((D, 3 * D), lambda i: (0, 0)),
        pl.BlockSpec((1, 3 * D), lambda i: (0, 0)),
        pl.BlockSpec((1, 3 * D), lambda i: (0, 0)),
    ],
    out_specs=pl.BlockSpec((NB, D), lambda i: (i, 0)),
    out_shape=jax.ShapeDtypeStruct((NP, D), jnp.float32),
)


def _s2s_body(out_ref, bcol_ref, brow_ref, wq_ref, wr_ref, whh_ref,
              bih_ref, bhh_ref, w1a_ref, w1b_ref, b1_ref, w2_ref, b2_ref,
              o_ref, q_scr, cs_scr, den_scr, rvn_scr, e_scr, emax_scr):
    it = pl.program_id(0)
    p = pl.program_id(1)
    k = pl.program_id(2)

    @pl.when((it == 0) & (p == 0) & (k == 0))
    def _():
        q_scr[...] = jnp.zeros((G, D), jnp.float32)
        cs_scr[...] = jnp.zeros((G, D), jnp.float32)
        rvn_scr[...] = jnp.zeros((G, D), jnp.float32)
        den_scr[...] = jnp.ones((G, 1), jnp.float32)

    @pl.when((p == 0) & (k == 0))
    def _():
        rvec = rvn_scr[...] / (den_scr[...] + 1e-16)
        gates = (q_scr[...] @ wq_ref[...] + rvec @ wr_ref[...]
                 + q_scr[...] @ whh_ref[...] + bih_ref[...] + bhh_ref[...])
        ig = _sigmoid(gates[:, 0:D])
        fg = _sigmoid(gates[:, D:2 * D])
        gg = jnp.tanh(gates[:, 2 * D:3 * D])
        og = _sigmoid(gates[:, 3 * D:4 * D])
        cs = fg * cs_scr[...] + ig * gg
        hs = og * jnp.tanh(cs)
        q_scr[...] = hs
        cs_scr[...] = cs
        emax_scr[...] = jnp.full((1, G), -jnp.inf, jnp.float32)

    S = (bcol_ref[...] == lax.broadcasted_iota(
        jnp.int32, (NB, G), 1)).astype(jnp.float32)

    @pl.when(p == 0)
    def _():
        qe = S @ q_scr[...]
        e = jnp.sum(out_ref[...] * qe, axis=1, keepdims=True)
        e_scr[pl.ds(k * NB, NB), :] = e
        colvals = jnp.where(S > 0.5, e, -jnp.inf)
        pmax = jnp.max(colvals, axis=0, keepdims=True)
        emax_scr[...] = jnp.maximum(emax_scr[...], pmax)

    @pl.when(p == 1)
    def _():
        em = emax_scr[...]
        em = jnp.where(jnp.isfinite(em), em, 0.0)
        eexp = jnp.sum(S * em, axis=1, keepdims=True)
        ee = jnp.exp(e_scr[pl.ds(k * NB, NB), :] - eexp)
        ST = (brow_ref[...] == lax.broadcasted_iota(
            jnp.int32, (G, NB), 0)).astype(jnp.float32)

        @pl.when(k == 0)
        def _():
            den_scr[...] = jnp.zeros((G, 1), jnp.float32)
            rvn_scr[...] = jnp.zeros((G, D), jnp.float32)

        den_scr[...] += ST @ ee
        rvn_scr[...] += ST @ (ee * out_ref[...])

    @pl.when((it == 2) & (p == 1) & (k == NP // NB - 1))
    def _():
        rvec = rvn_scr[...] / (den_scr[...] + 1e-16)
        t = jnp.maximum(q_scr[...] @ w1a_ref[...] + rvec @ w1b_ref[...]
                        + b1_ref[...], 0.0)
        o_ref[...] = t @ w2_ref[...] + b2_ref[...]


_s2s = pl.pallas_call(
    _s2s_body,
    grid=(3, 2, NP // NB),
    in_specs=[
        pl.BlockSpec((NB, D), lambda it, p, k: (k, 0)),
        pl.BlockSpec((NB, 1), lambda it, p, k: (k, 0)),
        pl.BlockSpec((1, NB), lambda it, p, k: (0, k)),
        pl.BlockSpec((D, 4 * D), lambda it, p, k: (0, 0)),
        pl.BlockSpec((D, 4 * D), lambda it, p, k: (0, 0)),
        pl.BlockSpec((D, 4 * D), lambda it, p, k: (0, 0)),
        pl.BlockSpec((1, 4 * D), lambda it, p, k: (0, 0)),
        pl.BlockSpec((1, 4 * D), lambda it, p, k: (0, 0)),
        pl.BlockSpec((D, D), lambda it, p, k: (0, 0)),
        pl.BlockSpec((D, D), lambda it, p, k: (0, 0)),
        pl.BlockSpec((1, D), lambda it, p, k: (0, 0)),
        pl.BlockSpec((D, 1), lambda it, p, k: (0, 0)),
        pl.BlockSpec((1, 1), lambda it, p, k: (0, 0)),
    ],
    out_specs=pl.BlockSpec((G, 1), lambda it, p, k: (0, 0)),
    out_shape=jax.ShapeDtypeStruct((G, 1), jnp.float32),
    scratch_shapes=[
        pltpu.VMEM((G, D), jnp.float32),
        pltpu.VMEM((G, D), jnp.float32),
        pltpu.VMEM((G, 1), jnp.float32),
        pltpu.VMEM((G, D), jnp.float32),
        pltpu.VMEM((NP, 1), jnp.float32),
        pltpu.VMEM((1, G), jnp.float32),
    ],
)


def kernel(x, edge_index, edge_attr, batch, lin0_W, lin0_b, nn1_W, nn1_b,
           nn2_W, nn2_b, root_W, conv_b, gru_W_ih, gru_W_hh, gru_b_ih,
           gru_b_hh, lstm_W_ih, lstm_W_hh, lstm_b_ih, lstm_b_hh, lin1_W,
           lin1_b, lin2_W, lin2_b):
    f32 = jnp.float32
    xp = jnp.zeros((NP, FEAT), f32).at[:N].set(x)
    src = jnp.zeros((EP,), jnp.int32).at[:E].set(edge_index[0])
    dst = jnp.full((EP,), NP - 1, jnp.int32).at[:E].set(edge_index[1])
    src3 = src.reshape(NW, NCH, CH)
    dst3 = dst.reshape(NW, NCH, CH)
    ea8 = jnp.zeros((EP, 8), f32).at[:E, :5].set(edge_attr)
    batchp = jnp.full((NP,), G - 1, jnp.int32).at[:N].set(batch)
    bcol = batchp.reshape(NP, 1)
    brow = batchp.reshape(1, NP)
    w1p = jnp.zeros((8, 128), f32).at[:5].set(nn1_W)
    z64 = jnp.zeros((NP, D), f32)
    z16 = jnp.zeros((NP, 16), f32)
    ones_ch = jnp.ones((CH, 16), f32)
    lane = jnp.arange(4096, dtype=jnp.int32)
    rmat = (lane[None, :] // D == jnp.arange(D, dtype=jnp.int32)[:, None]
            ).astype(f32)
    mmat = (lane[:512, None] % D == jnp.arange(D, dtype=jnp.int32)[None, :]
            ).astype(f32)

    sc_gather, sc_scatter_deg, sc_scatter = _sc_kernels()
    h = _lin0(xp, lin0_W, lin0_b.reshape(1, D))
    degp = None
    for it in range(3):
        xs = sc_gather(h, src3)
        msg = _msg(ea8, xs, w1p, nn1_b.reshape(1, 128), nn2_W,
                   nn2_b.reshape(1, 4096), rmat, mmat)
        if it == 0:
            aggp, degp = sc_scatter_deg(msg, dst3, z64, z16, ones_ch)
        else:
            aggp = sc_scatter(msg, dst3, z64)
        h = _gru(h, aggp[0], aggp[1], degp[0], degp[1], root_W,
                 conv_b.reshape(1, D), gru_W_ih, gru_W_hh,
                 gru_b_ih.reshape(1, 3 * D), gru_b_hh.reshape(1, 3 * D))

    out = h
    wq = lstm_W_ih[:D]
    wr = lstm_W_ih[D:]
    o = _s2s(out, bcol, brow, wq, wr, lstm_W_hh,
             lstm_b_ih.reshape(1, 4 * D), lstm_b_hh.reshape(1, 4 * D),
             lin1_W[:D], lin1_W[D:], lin1_b.reshape(1, D), lin2_W,
             lin2_b.reshape(1, 1))
    return o[:NG, 0]


# speedup vs baseline: 2.6184x; 1.0347x over previous
import functools

import jax
import jax.numpy as jnp
from jax import lax
from jax.experimental import pallas as pl
from jax.experimental.pallas import tpu as pltpu
from jax.experimental.pallas import tpu_sc as plsc

N = 10000
E = 20000
FEAT = 16
D = 64
NG = 500

NP = 10240
EP = 20480
G = 512
NC = 2
NS = 16
NW = NC * NS
CH = 128
NCH = EP // (NW * CH)
RPT = NP // NS

EB = 2048
NB = 512

def _sc_gather_body(nodes_hbm, src_hbm, xs_hbm, idx_v, row_v, gsem, wsem):
    c = lax.axis_index("c")
    s = lax.axis_index("s")
    wid = c * NS + s
    pltpu.sync_copy(src_hbm.at[wid], idx_v)
    base = wid * NCH * CH
    gd = [pltpu.async_copy(nodes_hbm.at[idx_v.at[j]], row_v.at[j], gsem)
          for j in range(NCH)]
    for d in gd:
        d.wait()
    wd = [pltpu.async_copy(row_v.at[j],
                           xs_hbm.at[pl.ds(base + j * CH, CH)], wsem)
          for j in range(NCH)]
    for d in wd:
        d.wait()


def _scatter_body(with_deg, msg_hbm, dst_hbm, z64_hbm, z16_hbm, ones_hbm,
                  agg_hbm, deg_hbm, idx_v, msg_v, ones_v, spA, spD,
                  lsem, ssem):
    c = lax.axis_index("c")
    s = lax.axis_index("s")
    wid = c * NS + s
    rows = pl.ds(s * RPT, RPT)
    base = wid * NCH * CH
    pltpu.sync_copy(dst_hbm.at[wid], idx_v)
    ld = [pltpu.async_copy(msg_hbm.at[pl.ds(base + j * CH, CH)],
                           msg_v.at[j], lsem)
          for j in range(NCH)]
    pltpu.sync_copy(z64_hbm.at[rows], spA.at[rows])
    if with_deg:
        pltpu.sync_copy(z16_hbm.at[rows], spD.at[rows])
        pltpu.sync_copy(ones_hbm, ones_v)
    for d in ld:
        d.wait()
    plsc.subcore_barrier()
    sd = [pltpu.async_copy(msg_v.at[j], spA.at[idx_v.at[j]], ssem, add=True)
          for j in range(NCH)]
    if with_deg:
        sd += [pltpu.async_copy(ones_v, spD.at[idx_v.at[j]], ssem, add=True)
               for j in range(NCH)]
    for d in sd:
        d.wait()
    plsc.subcore_barrier()
    pltpu.sync_copy(spA.at[rows], agg_hbm.at[c, rows])
    if with_deg:
        pltpu.sync_copy(spD.at[rows], deg_hbm.at[c, rows])


def _scatter_nodeg_body(msg_hbm, dst_hbm, z64_hbm, agg_hbm,
                        idx_v, msg_v, spA, lsem, ssem):
    _scatter_body(False, msg_hbm, dst_hbm, z64_hbm, None, None,
                  agg_hbm, None, idx_v, msg_v, None, spA, None, lsem, ssem)


@functools.cache
def _sc_kernels():
    mesh = plsc.VectorSubcoreMesh(
        core_axis_name="c", subcore_axis_name="s",
        num_cores=NC, num_subcores=NS)
    cp = pltpu.CompilerParams(use_tc_tiling_on_sc=False)
    gather = pl.kernel(
        _sc_gather_body,
        out_type=jax.ShapeDtypeStruct((EP, D), jnp.float32),
        mesh=mesh,
        scratch_types=[
            pltpu.VMEM((NCH, CH), jnp.int32),
            pltpu.VMEM((NCH, CH, D), jnp.float32),
            pltpu.SemaphoreType.DMA,
            pltpu.SemaphoreType.DMA,
        ],
        compiler_params=cp,
    )
    scatter_deg = pl.kernel(
        functools.partial(_scatter_body, True),
        out_type=(jax.ShapeDtypeStruct((NC, NP, D), jnp.float32),
                  jax.ShapeDtypeStruct((NC, NP, 16), jnp.float32)),
        mesh=mesh,
        scratch_types=[
            pltpu.VMEM((NCH, CH), jnp.int32),
            pltpu.VMEM((NCH, CH, D), jnp.float32),
            pltpu.VMEM((CH, 16), jnp.float32),
            pltpu.VMEM_SHARED((NP, D), jnp.float32),
            pltpu.VMEM_SHARED((NP, 16), jnp.float32),
            pltpu.SemaphoreType.DMA,
            pltpu.SemaphoreType.DMA,
        ],
        compiler_params=cp,
    )
    scatter = pl.kernel(
        _scatter_nodeg_body,
        out_type=jax.ShapeDtypeStruct((NC, NP, D), jnp.float32),
        mesh=mesh,
        scratch_types=[
            pltpu.VMEM((NCH, CH), jnp.int32),
            pltpu.VMEM((NCH, CH, D), jnp.float32),
            pltpu.VMEM_SHARED((NP, D), jnp.float32),
            pltpu.SemaphoreType.DMA,
            pltpu.SemaphoreType.DMA,
        ],
        compiler_params=cp,
    )
    return gather, scatter_deg, scatter


def _lin0_body(x_ref, w_ref, b_ref, o_ref):
    o_ref[...] = jnp.maximum(x_ref[...] @ w_ref[...] + b_ref[...], 0.0)


_lin0 = pl.pallas_call(
    _lin0_body,
    grid=(NP // NB,),
    in_specs=[
        pl.BlockSpec((NB, FEAT), lambda i: (i, 0)),
        pl.BlockSpec((FEAT, D), lambda i: (0, 0)),
        pl.BlockSpec((1, D), lambda i: (0, 0)),
    ],
    out_specs=pl.BlockSpec((NB, D), lambda i: (i, 0)),
    out_shape=jax.ShapeDtypeStruct((NP, D), jnp.float32),
)


def _msg_body(ea_ref, xs_ref, w1_ref, b1_ref, w2_ref, b2_ref, r_ref, m_ref,
              o_ref, h2_ref, ew_ref):
    h2_ref[...] = jnp.maximum(ea_ref[...] @ w1_ref[...] + b1_ref[...], 0.0)
    for cch in range(8):
        ew = (h2_ref[...] @ w2_ref[:, cch * 512:(cch + 1) * 512]
              + b2_ref[:, cch * 512:(cch + 1) * 512])
        xb = xs_ref[...] @ r_ref[:, cch * 512:(cch + 1) * 512]
        if cch == 0:
            ew_ref[...] = ew * xb
        else:
            ew_ref[...] += ew * xb
    o_ref[...] = ew_ref[...] @ m_ref[...]


_msg = pl.pallas_call(
    _msg_body,
    grid=(EP // EB,),
    in_specs=[
        pl.BlockSpec((EB, 8), lambda i: (i, 0)),
        pl.BlockSpec((EB, D), lambda i: (i, 0)),
        pl.BlockSpec((8, 128), lambda i: (0, 0)),
        pl.BlockSpec((1, 128), lambda i: (0, 0)),
        pl.BlockSpec((128, 4096), lambda i: (0, 0)),
        pl.BlockSpec((1, 4096), lambda i: (0, 0)),
        pl.BlockSpec((D, 4096), lambda i: (0, 0)),
        pl.BlockSpec((512, D), lambda i: (0, 0)),
    ],
    out_specs=pl.BlockSpec((EB, D), lambda i: (i, 0)),
    out_shape=jax.ShapeDtypeStruct((EP, D), jnp.float32),
    scratch_shapes=[
        pltpu.VMEM((EB, 128), jnp.float32),
        pltpu.VMEM((EB, 512), jnp.float32),
    ],
)


def _sigmoid(x):
    return 1.0 / (1.0 + jnp.exp(-x))


def _gru_body(h_ref, a0_ref, a1_ref, d0_ref, d1_ref, rw_ref, cb_ref,
              wih_ref, whh_ref, bih_ref, bhh_ref, o_ref):
    h = h_ref[...]
    agg = a0_ref[...] + a1_ref[...]
    deg = jnp.maximum(d0_ref[:, 0:1] + d1_ref[:, 0:1], 1.0)
    m = jnp.maximum(agg / deg + h @ rw_ref[...] + cb_ref[...], 0.0)
    gi = m @ wih_ref[...] + bih_ref[...]
    gh = h @ whh_ref[...] + bhh_ref[...]
    r = _sigmoid(gi[:, 0:D] + gh[:, 0:D])
    z = _sigmoid(gi[:, D:2 * D] + gh[:, D:2 * D])
    cand = jnp.tanh(gi[:, 2 * D:3 * D] + r * gh[:, 2 * D:3 * D])
    o_ref[...] = (1.0 - z) * cand + z * h


_gru = pl.pallas_call(
    _gru_body,
    grid=(NP // NB,),
    in_specs=[
        pl.BlockSpec((NB, D), lambda i: (i, 0)),
        pl.BlockSpec((NB, D), lambda i: (i, 0)),
        pl.BlockSpec((NB, D), lambda i: (i, 0)),
        pl.BlockSpec((NB, 16), lambda i: (i, 0)),
        pl.BlockSpec((NB, 16), lambda i: (i, 0)),
        pl.BlockSpec((D, D), lambda i: (0, 0)),
        pl.BlockSpec((1, D), lambda i: (0, 0)),
        pl.BlockSpec((D, 3 * D), lambda i: (0, 0)),
        pl.BlockSpec((D, 3 * D), lambda i: (0, 0)),
        pl.BlockSpec((1, 3 * D), lambda i: (0, 0)),
        pl.BlockSpec((1, 3 * D), lambda i: (0, 0)),
    ],
    out_specs=pl.BlockSpec((NB, D), lambda i: (i, 0)),
    out_shape=jax.ShapeDtypeStruct((NP, D), jnp.float32),
)


def _s2s_body(out_ref, bcol_ref, brow_ref, wq_ref, wr_ref, whh_ref,
              bih_ref, bhh_ref, w1a_ref, w1b_ref, b1_ref, w2_ref, b2_ref,
              o_ref, q_scr, cs_scr, den_scr, rvn_scr, e_scr, emax_scr):
    it = pl.program_id(0)
    p = pl.program_id(1)
    k = pl.program_id(2)

    @pl.when((it == 0) & (p == 0) & (k == 0))
    def _():
        q_scr[...] = jnp.zeros((G, D), jnp.float32)
        cs_scr[...] = jnp.zeros((G, D), jnp.float32)
        rvn_scr[...] = jnp.zeros((G, D), jnp.float32)
        den_scr[...] = jnp.ones((G, 1), jnp.float32)

    @pl.when((p == 0) & (k == 0))
    def _():
        rvec = rvn_scr[...] / (den_scr[...] + 1e-16)
        gates = (q_scr[...] @ wq_ref[...] + rvec @ wr_ref[...]
                 + q_scr[...] @ whh_ref[...] + bih_ref[...] + bhh_ref[...])
        ig = _sigmoid(gates[:, 0:D])
        fg = _sigmoid(gates[:, D:2 * D])
        gg = jnp.tanh(gates[:, 2 * D:3 * D])
        og = _sigmoid(gates[:, 3 * D:4 * D])
        cs = fg * cs_scr[...] + ig * gg
        hs = og * jnp.tanh(cs)
        q_scr[...] = hs
        cs_scr[...] = cs
        emax_scr[...] = jnp.full((1, G), -jnp.inf, jnp.float32)

    S = (bcol_ref[...] == lax.broadcasted_iota(
        jnp.int32, (NB, G), 1)).astype(jnp.float32)

    @pl.when(p == 0)
    def _():
        qe = S @ q_scr[...]
        e = jnp.sum(out_ref[...] * qe, axis=1, keepdims=True)
        e_scr[pl.ds(k * NB, NB), :] = e
        colvals = jnp.where(S > 0.5, e, -jnp.inf)
        pmax = jnp.max(colvals, axis=0, keepdims=True)
        emax_scr[...] = jnp.maximum(emax_scr[...], pmax)

    @pl.when(p == 1)
    def _():
        em = emax_scr[...]
        em = jnp.where(jnp.isfinite(em), em, 0.0)
        eexp = jnp.sum(S * em, axis=1, keepdims=True)
        ee = jnp.exp(e_scr[pl.ds(k * NB, NB), :] - eexp)
        ST = (brow_ref[...] == lax.broadcasted_iota(
            jnp.int32, (G, NB), 0)).astype(jnp.float32)

        @pl.when(k == 0)
        def _():
            den_scr[...] = jnp.zeros((G, 1), jnp.float32)
            rvn_scr[...] = jnp.zeros((G, D), jnp.float32)

        den_scr[...] += ST @ ee
        rvn_scr[...] += ST @ (ee * out_ref[...])

    @pl.when((it == 2) & (p == 1) & (k == NP // NB - 1))
    def _():
        rvec = rvn_scr[...] / (den_scr[...] + 1e-16)
        t = jnp.maximum(q_scr[...] @ w1a_ref[...] + rvec @ w1b_ref[...]
                        + b1_ref[...], 0.0)
        o_ref[...] = t @ w2_ref[...] + b2_ref[...]


_s2s = pl.pallas_call(
    _s2s_body,
    grid=(3, 2, NP // NB),
    in_specs=[
        pl.BlockSpec((NB, D), lambda it, p, k: (k, 0)),
        pl.BlockSpec((NB, 1), lambda it, p, k: (k, 0)),
        pl.BlockSpec((1, NB), lambda it, p, k: (0, k)),
        pl.BlockSpec((D, 4 * D), lambda it, p, k: (0, 0)),
        pl.BlockSpec((D, 4 * D), lambda it, p, k: (0, 0)),
        pl.BlockSpec((D, 4 * D), lambda it, p, k: (0, 0)),
        pl.BlockSpec((1, 4 * D), lambda it, p, k: (0, 0)),
        pl.BlockSpec((1, 4 * D), lambda it, p, k: (0, 0)),
        pl.BlockSpec((D, D), lambda it, p, k: (0, 0)),
        pl.BlockSpec((D, D), lambda it, p, k: (0, 0)),
        pl.BlockSpec((1, D), lambda it, p, k: (0, 0)),
        pl.BlockSpec((D, 1), lambda it, p, k: (0, 0)),
        pl.BlockSpec((1, 1), lambda it, p, k: (0, 0)),
    ],
    out_specs=pl.BlockSpec((G, 1), lambda it, p, k: (0, 0)),
    out_shape=jax.ShapeDtypeStruct((G, 1), jnp.float32),
    scratch_shapes=[
        pltpu.VMEM((G, D), jnp.float32),
        pltpu.VMEM((G, D), jnp.float32),
        pltpu.VMEM((G, 1), jnp.float32),
        pltpu.VMEM((G, D), jnp.float32),
        pltpu.VMEM((NP, 1), jnp.float32),
        pltpu.VMEM((1, G), jnp.float32),
    ],
)


def kernel(x, edge_index, edge_attr, batch, lin0_W, lin0_b, nn1_W, nn1_b,
           nn2_W, nn2_b, root_W, conv_b, gru_W_ih, gru_W_hh, gru_b_ih,
           gru_b_hh, lstm_W_ih, lstm_W_hh, lstm_b_ih, lstm_b_hh, lin1_W,
           lin1_b, lin2_W, lin2_b):
    f32 = jnp.float32
    xp = jnp.zeros((NP, FEAT), f32).at[:N].set(x)
    src = jnp.zeros((EP,), jnp.int32).at[:E].set(edge_index[0])
    dst = jnp.full((EP,), NP - 1, jnp.int32).at[:E].set(edge_index[1])
    src3 = src.reshape(NW, NCH, CH)
    dst3 = dst.reshape(NW, NCH, CH)
    ea8 = jnp.zeros((EP, 8), f32).at[:E, :5].set(edge_attr)
    batchp = jnp.full((NP,), G - 1, jnp.int32).at[:N].set(batch)
    bcol = batchp.reshape(NP, 1)
    brow = batchp.reshape(1, NP)
    w1p = jnp.zeros((8, 128), f32).at[:5].set(nn1_W)
    z64 = jnp.zeros((NP, D), f32)
    z16 = jnp.zeros((NP, 16), f32)
    ones_ch = jnp.ones((CH, 16), f32)
    lane = jnp.arange(4096, dtype=jnp.int32)
    rmat = (lane[None, :] // D == jnp.arange(D, dtype=jnp.int32)[:, None]
            ).astype(f32)
    mmat = (lane[:512, None] % D == jnp.arange(D, dtype=jnp.int32)[None, :]
            ).astype(f32)

    sc_gather, sc_scatter_deg, sc_scatter = _sc_kernels()
    h = _lin0(xp, lin0_W, lin0_b.reshape(1, D))
    degp = None
    for it in range(3):
        xs = sc_gather(h, src3)
        msg = _msg(ea8, xs, w1p, nn1_b.reshape(1, 128), nn2_W,
                   nn2_b.reshape(1, 4096), rmat, mmat)
        if it == 0:
            aggp, degp = sc_scatter_deg(msg, dst3, z64, z16, ones_ch)
        else:
            aggp = sc_scatter(msg, dst3, z64)
        h = _gru(h, aggp[0], aggp[1], degp[0], degp[1], root_W,
                 conv_b.reshape(1, D), gru_W_ih, gru_W_hh,
                 gru_b_ih.reshape(1, 3 * D), gru_b_hh.reshape(1, 3 * D))

    out = h
    wq = lstm_W_ih[:D]
    wr = lstm_W_ih[D:]
    o = _s2s(out, bcol, brow, wq, wr, lstm_W_hh,
             lstm_b_ih.reshape(1, 4 * D), lstm_b_hh.reshape(1, 4 * D),
             lin1_W[:D], lin1_W[D:], lin1_b.reshape(1, D), lin2_W,
             lin2_b.reshape(1, 1))
    return o[:NG, 0]


# speedup vs baseline: 2.9065x; 1.1101x over previous
import functools

import jax
import jax.numpy as jnp
from jax import lax
from jax.experimental import pallas as pl
from jax.experimental.pallas import tpu as pltpu
from jax.experimental.pallas import tpu_sc as plsc

N = 10000
E = 20000
FEAT = 16
D = 64
NG = 500

NP = 10240
EP = 20480
G = 512
NC = 2
NS = 16
NW = NC * NS
CH = 128
NCH = EP // (NW * CH)
RPT = NP // NS

EB = 2048
NB = 1024
NBG = 2048

def _sc_gather_body(nodes_hbm, src_hbm, xs_hbm, idx_v, row_v, gsem, wsem):
    c = lax.axis_index("c")
    s = lax.axis_index("s")
    wid = c * NS + s
    pltpu.sync_copy(src_hbm.at[wid], idx_v)
    base = wid * NCH * CH
    gd = [pltpu.async_copy(nodes_hbm.at[idx_v.at[j]], row_v.at[j], gsem)
          for j in range(NCH)]
    for d in gd:
        d.wait()
    wd = [pltpu.async_copy(row_v.at[j],
                           xs_hbm.at[pl.ds(base + j * CH, CH)], wsem)
          for j in range(NCH)]
    for d in wd:
        d.wait()


def _scatter_body(with_deg, msg_hbm, dst_hbm, z64_hbm, z16_hbm, ones_hbm,
                  agg_hbm, deg_hbm, idx_v, msg_v, ones_v, spA, spD,
                  lsem, ssem):
    c = lax.axis_index("c")
    s = lax.axis_index("s")
    wid = c * NS + s
    rows = pl.ds(s * RPT, RPT)
    base = wid * NCH * CH
    pltpu.sync_copy(dst_hbm.at[wid], idx_v)
    ld = [pltpu.async_copy(msg_hbm.at[pl.ds(base + j * CH, CH)],
                           msg_v.at[j], lsem)
          for j in range(NCH)]
    pltpu.sync_copy(z64_hbm.at[rows], spA.at[rows])
    if with_deg:
        pltpu.sync_copy(z16_hbm.at[rows], spD.at[rows])
        pltpu.sync_copy(ones_hbm, ones_v)
    for d in ld:
        d.wait()
    plsc.subcore_barrier()
    sd = [pltpu.async_copy(msg_v.at[j], spA.at[idx_v.at[j]], ssem, add=True)
          for j in range(NCH)]
    if with_deg:
        sd += [pltpu.async_copy(ones_v, spD.at[idx_v.at[j]], ssem, add=True)
               for j in range(NCH)]
    for d in sd:
        d.wait()
    plsc.subcore_barrier()
    pltpu.sync_copy(spA.at[rows], agg_hbm.at[c, rows])
    if with_deg:
        pltpu.sync_copy(spD.at[rows], deg_hbm.at[c, rows])


def _scatter_nodeg_body(msg_hbm, dst_hbm, z64_hbm, agg_hbm,
                        idx_v, msg_v, spA, lsem, ssem):
    _scatter_body(False, msg_hbm, dst_hbm, z64_hbm, None, None,
                  agg_hbm, None, idx_v, msg_v, None, spA, None, lsem, ssem)


@functools.cache
def _sc_kernels():
    mesh = plsc.VectorSubcoreMesh(
        core_axis_name="c", subcore_axis_name="s",
        num_cores=NC, num_subcores=NS)
    cp = pltpu.CompilerParams(use_tc_tiling_on_sc=False)
    gather = pl.kernel(
        _sc_gather_body,
        out_type=jax.ShapeDtypeStruct((EP, D), jnp.float32),
        mesh=mesh,
        scratch_types=[
            pltpu.VMEM((NCH, CH), jnp.int32),
            pltpu.VMEM((NCH, CH, D), jnp.float32),
            pltpu.SemaphoreType.DMA,
            pltpu.SemaphoreType.DMA,
        ],
        compiler_params=cp,
    )
    gather16 = pl.kernel(
        _sc_gather_body,
        out_type=jax.ShapeDtypeStruct((EP, FEAT), jnp.float32),
        mesh=mesh,
        scratch_types=[
            pltpu.VMEM((NCH, CH), jnp.int32),
            pltpu.VMEM((NCH, CH, FEAT), jnp.float32),
            pltpu.SemaphoreType.DMA,
            pltpu.SemaphoreType.DMA,
        ],
        compiler_params=cp,
    )
    scatter_deg = pl.kernel(
        functools.partial(_scatter_body, True),
        out_type=(jax.ShapeDtypeStruct((NC, NP, D), jnp.float32),
                  jax.ShapeDtypeStruct((NC, NP, 16), jnp.float32)),
        mesh=mesh,
        scratch_types=[
            pltpu.VMEM((NCH, CH), jnp.int32),
            pltpu.VMEM((NCH, CH, D), jnp.float32),
            pltpu.VMEM((CH, 16), jnp.float32),
            pltpu.VMEM_SHARED((NP, D), jnp.float32),
            pltpu.VMEM_SHARED((NP, 16), jnp.float32),
            pltpu.SemaphoreType.DMA,
            pltpu.SemaphoreType.DMA,
        ],
        compiler_params=cp,
    )
    scatter = pl.kernel(
        _scatter_nodeg_body,
        out_type=jax.ShapeDtypeStruct((NC, NP, D), jnp.float32),
        mesh=mesh,
        scratch_types=[
            pltpu.VMEM((NCH, CH), jnp.int32),
            pltpu.VMEM((NCH, CH, D), jnp.float32),
            pltpu.VMEM_SHARED((NP, D), jnp.float32),
            pltpu.SemaphoreType.DMA,
            pltpu.SemaphoreType.DMA,
        ],
        compiler_params=cp,
    )
    return gather, gather16, scatter_deg, scatter


def _lin0_body(x_ref, w_ref, b_ref, o_ref):
    o_ref[...] = jnp.maximum(x_ref[...] @ w_ref[...] + b_ref[...], 0.0)


_lin0 = pl.pallas_call(
    _lin0_body,
    grid=(NP // NBG,),
    in_specs=[
        pl.BlockSpec((NBG, FEAT), lambda i: (i, 0)),
        pl.BlockSpec((FEAT, D), lambda i: (0, 0)),
        pl.BlockSpec((1, D), lambda i: (0, 0)),
    ],
    out_specs=pl.BlockSpec((NBG, D), lambda i: (i, 0)),
    out_shape=jax.ShapeDtypeStruct((NP, D), jnp.float32),
)


def _msg_body(ea_ref, xs_ref, w1_ref, b1_ref, w2_ref, b2_ref, r_ref, m_ref,
              o_ref, h2_ref, ew_ref):
    h2_ref[...] = jnp.maximum(ea_ref[...] @ w1_ref[...] + b1_ref[...], 0.0)
    for cch in range(8):
        ew = (h2_ref[...] @ w2_ref[:, cch * 512:(cch + 1) * 512]
              + b2_ref[:, cch * 512:(cch + 1) * 512])
        xb = xs_ref[...] @ r_ref[:, cch * 512:(cch + 1) * 512]
        if cch == 0:
            ew_ref[...] = ew * xb
        else:
            ew_ref[...] += ew * xb
    o_ref[...] = ew_ref[...] @ m_ref[...]


_msg = pl.pallas_call(
    _msg_body,
    grid=(EP // EB,),
    in_specs=[
        pl.BlockSpec((EB, 8), lambda i: (i, 0)),
        pl.BlockSpec((EB, D), lambda i: (i, 0)),
        pl.BlockSpec((8, 128), lambda i: (0, 0)),
        pl.BlockSpec((1, 128), lambda i: (0, 0)),
        pl.BlockSpec((128, 4096), lambda i: (0, 0)),
        pl.BlockSpec((1, 4096), lambda i: (0, 0)),
        pl.BlockSpec((D, 4096), lambda i: (0, 0)),
        pl.BlockSpec((512, D), lambda i: (0, 0)),
    ],
    out_specs=pl.BlockSpec((EB, D), lambda i: (i, 0)),
    out_shape=jax.ShapeDtypeStruct((EP, D), jnp.float32),
    scratch_shapes=[
        pltpu.VMEM((EB, 128), jnp.float32),
        pltpu.VMEM((EB, 512), jnp.float32),
    ],
)


def _msg1_body(ea_ref, xg_ref, w0_ref, b0_ref, w1_ref, b1_ref, w2_ref,
               b2_ref, r_ref, m_ref, o_ref, h2_ref, ew_ref):
    xs = jnp.maximum(xg_ref[...] @ w0_ref[...] + b0_ref[...], 0.0)
    h2_ref[...] = jnp.maximum(ea_ref[...] @ w1_ref[...] + b1_ref[...], 0.0)
    for cch in range(8):
        ew = (h2_ref[...] @ w2_ref[:, cch * 512:(cch + 1) * 512]
              + b2_ref[:, cch * 512:(cch + 1) * 512])
        xb = xs @ r_ref[:, cch * 512:(cch + 1) * 512]
        if cch == 0:
            ew_ref[...] = ew * xb
        else:
            ew_ref[...] += ew * xb
    o_ref[...] = ew_ref[...] @ m_ref[...]


_msg1 = pl.pallas_call(
    _msg1_body,
    grid=(EP // EB,),
    in_specs=[
        pl.BlockSpec((EB, 8), lambda i: (i, 0)),
        pl.BlockSpec((EB, FEAT), lambda i: (i, 0)),
        pl.BlockSpec((FEAT, D), lambda i: (0, 0)),
        pl.BlockSpec((1, D), lambda i: (0, 0)),
        pl.BlockSpec((8, 128), lambda i: (0, 0)),
        pl.BlockSpec((1, 128), lambda i: (0, 0)),
        pl.BlockSpec((128, 4096), lambda i: (0, 0)),
        pl.BlockSpec((1, 4096), lambda i: (0, 0)),
        pl.BlockSpec((D, 4096), lambda i: (0, 0)),
        pl.BlockSpec((512, D), lambda i: (0, 0)),
    ],
    out_specs=pl.BlockSpec((EB, D), lambda i: (i, 0)),
    out_shape=jax.ShapeDtypeStruct((EP, D), jnp.float32),
    scratch_shapes=[
        pltpu.VMEM((EB, 128), jnp.float32),
        pltpu.VMEM((EB, 512), jnp.float32),
    ],
)


def _sigmoid(x):
    return 1.0 / (1.0 + jnp.exp(-x))


def _gru_body(h_ref, a0_ref, a1_ref, d0_ref, d1_ref, rw_ref, cb_ref,
              wih_ref, whh_ref, bih_ref, bhh_ref, o_ref):
    h = h_ref[...]
    agg = a0_ref[...] + a1_ref[...]
    deg = jnp.maximum(d0_ref[:, 0:1] + d1_ref[:, 0:1], 1.0)
    m = jnp.maximum(agg / deg + h @ rw_ref[...] + cb_ref[...], 0.0)
    gi = m @ wih_ref[...] + bih_ref[...]
    gh = h @ whh_ref[...] + bhh_ref[...]
    r = _sigmoid(gi[:, 0:D] + gh[:, 0:D])
    z = _sigmoid(gi[:, D:2 * D] + gh[:, D:2 * D])
    cand = jnp.tanh(gi[:, 2 * D:3 * D] + r * gh[:, 2 * D:3 * D])
    o_ref[...] = (1.0 - z) * cand + z * h


_gru = pl.pallas_call(
    _gru_body,
    grid=(NP // NBG,),
    in_specs=[
        pl.BlockSpec((NBG, D), lambda i: (i, 0)),
        pl.BlockSpec((NBG, D), lambda i: (i, 0)),
        pl.BlockSpec((NBG, D), lambda i: (i, 0)),
        pl.BlockSpec((NBG, 16), lambda i: (i, 0)),
        pl.BlockSpec((NBG, 16), lambda i: (i, 0)),
        pl.BlockSpec((D, D), lambda i: (0, 0)),
        pl.BlockSpec((1, D), lambda i: (0, 0)),
        pl.BlockSpec((D, 3 * D), lambda i: (0, 0)),
        pl.BlockSpec((D, 3 * D), lambda i: (0, 0)),
        pl.BlockSpec((1, 3 * D), lambda i: (0, 0)),
        pl.BlockSpec((1, 3 * D), lambda i: (0, 0)),
    ],
    out_specs=pl.BlockSpec((NBG, D), lambda i: (i, 0)),
    out_shape=jax.ShapeDtypeStruct((NP, D), jnp.float32),
)


def _s2s_body(out_ref, bcol_ref, brow_ref, wq_ref, wr_ref, whh_ref,
              bih_ref, bhh_ref, w1a_ref, w1b_ref, b1_ref, w2_ref, b2_ref,
              o_ref, q_scr, cs_scr, den_scr, rvn_scr, e_scr, emax_scr):
    it = pl.program_id(0)
    p = pl.program_id(1)
    k = pl.program_id(2)

    @pl.when((it == 0) & (p == 0) & (k == 0))
    def _():
        q_scr[...] = jnp.zeros((G, D), jnp.float32)
        cs_scr[...] = jnp.zeros((G, D), jnp.float32)
        rvn_scr[...] = jnp.zeros((G, D), jnp.float32)
        den_scr[...] = jnp.ones((G, 1), jnp.float32)

    @pl.when((p == 0) & (k == 0))
    def _():
        rvec = rvn_scr[...] / (den_scr[...] + 1e-16)
        gates = (q_scr[...] @ wq_ref[...] + rvec @ wr_ref[...]
                 + q_scr[...] @ whh_ref[...] + bih_ref[...] + bhh_ref[...])
        ig = _sigmoid(gates[:, 0:D])
        fg = _sigmoid(gates[:, D:2 * D])
        gg = jnp.tanh(gates[:, 2 * D:3 * D])
        og = _sigmoid(gates[:, 3 * D:4 * D])
        cs = fg * cs_scr[...] + ig * gg
        hs = og * jnp.tanh(cs)
        q_scr[...] = hs
        cs_scr[...] = cs
        emax_scr[...] = jnp.full((1, G), -jnp.inf, jnp.float32)

    S = (bcol_ref[...] == lax.broadcasted_iota(
        jnp.int32, (NB, G), 1)).astype(jnp.float32)

    @pl.when(p == 0)
    def _():
        qe = S @ q_scr[...]
        e = jnp.sum(out_ref[...] * qe, axis=1, keepdims=True)
        e_scr[pl.ds(k * NB, NB), :] = e
        colvals = jnp.where(S > 0.5, e, -jnp.inf)
        pmax = jnp.max(colvals, axis=0, keepdims=True)
        emax_scr[...] = jnp.maximum(emax_scr[...], pmax)

    @pl.when(p == 1)
    def _():
        em = emax_scr[...]
        em = jnp.where(jnp.isfinite(em), em, 0.0)
        eexp = jnp.sum(S * em, axis=1, keepdims=True)
        ee = jnp.exp(e_scr[pl.ds(k * NB, NB), :] - eexp)
        ST = (brow_ref[...] == lax.broadcasted_iota(
            jnp.int32, (G, NB), 0)).astype(jnp.float32)

        @pl.when(k == 0)
        def _():
            den_scr[...] = jnp.zeros((G, 1), jnp.float32)
            rvn_scr[...] = jnp.zeros((G, D), jnp.float32)

        den_scr[...] += ST @ ee
        rvn_scr[...] += ST @ (ee * out_ref[...])

    @pl.when((it == 2) & (p == 1) & (k == NP // NB - 1))
    def _():
        rvec = rvn_scr[...] / (den_scr[...] + 1e-16)
        t = jnp.maximum(q_scr[...] @ w1a_ref[...] + rvec @ w1b_ref[...]
                        + b1_ref[...], 0.0)
        o_ref[...] = t @ w2_ref[...] + b2_ref[...]


_s2s = pl.pallas_call(
    _s2s_body,
    grid=(3, 2, NP // NB),
    in_specs=[
        pl.BlockSpec((NB, D), lambda it, p, k: (k, 0)),
        pl.BlockSpec((NB, 1), lambda it, p, k: (k, 0)),
        pl.BlockSpec((1, NB), lambda it, p, k: (0, k)),
        pl.BlockSpec((D, 4 * D), lambda it, p, k: (0, 0)),
        pl.BlockSpec((D, 4 * D), lambda it, p, k: (0, 0)),
        pl.BlockSpec((D, 4 * D), lambda it, p, k: (0, 0)),
        pl.BlockSpec((1, 4 * D), lambda it, p, k: (0, 0)),
        pl.BlockSpec((1, 4 * D), lambda it, p, k: (0, 0)),
        pl.BlockSpec((D, D), lambda it, p, k: (0, 0)),
        pl.BlockSpec((D, D), lambda it, p, k: (0, 0)),
        pl.BlockSpec((1, D), lambda it, p, k: (0, 0)),
        pl.BlockSpec((D, 1), lambda it, p, k: (0, 0)),
        pl.BlockSpec((1, 1), lambda it, p, k: (0, 0)),
    ],
    out_specs=pl.BlockSpec((G, 1), lambda it, p, k: (0, 0)),
    out_shape=jax.ShapeDtypeStruct((G, 1), jnp.float32),
    scratch_shapes=[
        pltpu.VMEM((G, D), jnp.float32),
        pltpu.VMEM((G, D), jnp.float32),
        pltpu.VMEM((G, 1), jnp.float32),
        pltpu.VMEM((G, D), jnp.float32),
        pltpu.VMEM((NP, 1), jnp.float32),
        pltpu.VMEM((1, G), jnp.float32),
    ],
)


def kernel(x, edge_index, edge_attr, batch, lin0_W, lin0_b, nn1_W, nn1_b,
           nn2_W, nn2_b, root_W, conv_b, gru_W_ih, gru_W_hh, gru_b_ih,
           gru_b_hh, lstm_W_ih, lstm_W_hh, lstm_b_ih, lstm_b_hh, lin1_W,
           lin1_b, lin2_W, lin2_b):
    f32 = jnp.float32
    xp = jnp.zeros((NP, FEAT), f32).at[:N].set(x)
    src = jnp.zeros((EP,), jnp.int32).at[:E].set(edge_index[0])
    dst = jnp.full((EP,), NP - 1, jnp.int32).at[:E].set(edge_index[1])
    src3 = src.reshape(NW, NCH, CH)
    dst3 = dst.reshape(NW, NCH, CH)
    ea8 = jnp.zeros((EP, 8), f32).at[:E, :5].set(edge_attr)
    batchp = jnp.full((NP,), G - 1, jnp.int32).at[:N].set(batch)
    bcol = batchp.reshape(NP, 1)
    brow = batchp.reshape(1, NP)
    w1p = jnp.zeros((8, 128), f32).at[:5].set(nn1_W)
    z64 = jnp.zeros((NP, D), f32)
    z16 = jnp.zeros((NP, 16), f32)
    ones_ch = jnp.ones((CH, 16), f32)
    lane = jnp.arange(4096, dtype=jnp.int32)
    rmat = (lane[None, :] // D == jnp.arange(D, dtype=jnp.int32)[:, None]
            ).astype(f32)
    mmat = (lane[:512, None] % D == jnp.arange(D, dtype=jnp.int32)[None, :]
            ).astype(f32)

    sc_gather, sc_gather16, sc_scatter_deg, sc_scatter = _sc_kernels()
    h = _lin0(xp, lin0_W, lin0_b.reshape(1, D))
    degp = None
    for it in range(3):
        if it == 0:
            xg = sc_gather16(xp, src3)
            msg = _msg1(ea8, xg, lin0_W, lin0_b.reshape(1, D), w1p,
                        nn1_b.reshape(1, 128), nn2_W,
                        nn2_b.reshape(1, 4096), rmat, mmat)
            aggp, degp = sc_scatter_deg(msg, dst3, z64, z16, ones_ch)
        else:
            xs = sc_gather(h, src3)
            msg = _msg(ea8, xs, w1p, nn1_b.reshape(1, 128), nn2_W,
                       nn2_b.reshape(1, 4096), rmat, mmat)
            aggp = sc_scatter(msg, dst3, z64)
        h = _gru(h, aggp[0], aggp[1], degp[0], degp[1], root_W,
                 conv_b.reshape(1, D), gru_W_ih, gru_W_hh,
                 gru_b_ih.reshape(1, 3 * D), gru_b_hh.reshape(1, 3 * D))

    out = h
    wq = lstm_W_ih[:D]
    wr = lstm_W_ih[D:]
    o = _s2s(out, bcol, brow, wq, wr, lstm_W_hh,
             lstm_b_ih.reshape(1, 4 * D), lstm_b_hh.reshape(1, 4 * D),
             lin1_W[:D], lin1_W[D:], lin1_b.reshape(1, D), lin2_W,
             lin2_b.reshape(1, 1))
    return o[:NG, 0]
